# Initial kernel scaffold; baseline (speedup 1.0000x reference)
import jax
import jax.numpy as jnp
from jax.experimental import pallas as pl


def kernel(x, edge_index, W1, b1, gamma1, beta1, W2, b2):
    raise NotImplementedError("write your pallas kernel here")


# speedup vs baseline: 8.3649x; 8.3649x over previous
import functools

import jax
import jax.numpy as jnp
from jax import lax
from jax.experimental import pallas as pl
from jax.experimental.pallas import tpu as pltpu
from jax.experimental.pallas import tpu_sc as plsc

N = 10000
E = 320000
NPAD = 10240
NW = 32
EPW = 10240
K = 128
CH = EPW // K
RPT = NPAD // 16


def _zero_vmem(buf, rows, d):
    def zrow(i, _):
        def zcol(j, _):
            buf[i, pl.ds(j * 16, 16)] = jnp.zeros((16,), jnp.float32)
            return 0
        return lax.fori_loop(0, d // 16, zcol, 0)
    lax.fori_loop(0, rows, zrow, 0)


def _make_sc_deg():
    mesh = plsc.VectorSubcoreMesh(core_axis_name="c", subcore_axis_name="s")

    @functools.partial(
        pl.kernel, mesh=mesh,
        out_type=jax.ShapeDtypeStruct((2, NPAD, 128), jnp.float32),
        scratch_types=[
            pltpu.VMEM((CH, K), jnp.int32),
            pltpu.VMEM((K, 128), jnp.float32),
            pltpu.VMEM((K, 128), jnp.float32),
            pltpu.VMEM_SHARED((NPAD, 128), jnp.float32),
        ],
    )
    def deg_kernel(dst_hbm, out_hbm, dst_v, ones_v, zero_v, acc):
        cid = lax.axis_index("c")
        sid = lax.axis_index("s")
        wid = cid * 16 + sid
        pltpu.sync_copy(dst_hbm.at[wid], dst_v)

        def orow(i, _):
            def ocol(j, _):
                ones_v[i, pl.ds(j * 16, 16)] = jnp.ones((16,), jnp.float32)
                zero_v[i, pl.ds(j * 16, 16)] = jnp.zeros((16,), jnp.float32)
                return 0
            return lax.fori_loop(0, 8, ocol, 0)
        lax.fori_loop(0, K, orow, 0)
        base = sid * RPT
        for r in range(RPT // K):
            pltpu.sync_copy(zero_v, acc.at[pl.ds(base + r * K, K)])
        plsc.subcore_barrier()

        def chunk(c, _):
            pltpu.sync_copy(ones_v, acc.at[dst_v.at[c]], add=True)
            return 0
        lax.fori_loop(0, CH, chunk, 0)
        plsc.subcore_barrier()
        for r in range(RPT // K):
            sl = pl.ds(base + r * K, K)
            pltpu.sync_copy(acc.at[sl], out_hbm.at[cid, sl])

    return deg_kernel


def _make_sc_agg(d):
    mesh = plsc.VectorSubcoreMesh(core_axis_name="c", subcore_axis_name="s")

    @functools.partial(
        pl.kernel, mesh=mesh,
        out_type=jax.ShapeDtypeStruct((2, NPAD, d), jnp.float32),
        scratch_types=[
            pltpu.VMEM((CH, K), jnp.int32),
            pltpu.VMEM((CH, K), jnp.int32),
            pltpu.VMEM((K, d), jnp.float32),
            pltpu.VMEM_SHARED((NPAD, d), jnp.float32),
            pltpu.SemaphoreType.DMA,
        ],
    )
    def agg_kernel(u_hbm, src_hbm, dst_hbm, out_hbm,
                   src_v, dst_v, rows_v, acc, sem):
        cid = lax.axis_index("c")
        sid = lax.axis_index("s")
        wid = cid * 16 + sid
        pltpu.sync_copy(src_hbm.at[wid], src_v)
        pltpu.sync_copy(dst_hbm.at[wid], dst_v)
        _zero_vmem(rows_v, K, d)
        base = sid * RPT
        for r in range(RPT // K):
            pltpu.sync_copy(rows_v, acc.at[pl.ds(base + r * K, K)])
        plsc.subcore_barrier()

        def chunk(c, _):
            pltpu.async_copy(u_hbm.at[src_v.at[c]], rows_v, sem).wait()
            pltpu.sync_copy(rows_v, acc.at[dst_v.at[c]], add=True)
            return 0
        lax.fori_loop(0, CH, chunk, 0)
        plsc.subcore_barrier()
        for r in range(RPT // K):
            sl = pl.ds(base + r * K, K)
            pltpu.sync_copy(acc.at[sl], out_hbm.at[cid, sl])

    return agg_kernel


def _matmul1(x, w1):
    BR = 1000

    def body(x_ref, w_ref, o_ref):
        o_ref[...] = lax.dot_general(
            x_ref[...], w_ref[...], (((1,), (1,)), ((), ())),
            preferred_element_type=jnp.float32)

    return pl.pallas_call(
        body,
        grid=(N // BR,),
        in_specs=[pl.BlockSpec((BR, 128), lambda i: (i, 0)),
                  pl.BlockSpec((128, 128), lambda i: (0, 0))],
        out_specs=pl.BlockSpec((BR, 128), lambda i: (i, 0)),
        out_shape=jax.ShapeDtypeStruct((N, 128), jnp.float32),
    )(x, w1)


def _dinv_u1(d0, d1, h):
    BR = 1000

    def body(d0_ref, d1_ref, h_ref, dinv_ref, u_ref):
        deg = d0_ref[...] + d1_ref[...] + 1.0
        dinv = lax.rsqrt(deg)
        dinv_ref[...] = dinv
        u_ref[...] = h_ref[...] * dinv[:, 0:1]

    return pl.pallas_call(
        body,
        grid=(N // BR,),
        in_specs=[pl.BlockSpec((BR, 8), lambda i: (i, 0)),
                  pl.BlockSpec((BR, 8), lambda i: (i, 0)),
                  pl.BlockSpec((BR, 128), lambda i: (i, 0))],
        out_specs=[pl.BlockSpec((BR, 8), lambda i: (i, 0)),
                   pl.BlockSpec((BR, 128), lambda i: (i, 0))],
        out_shape=[jax.ShapeDtypeStruct((N, 8), jnp.float32),
                   jax.ShapeDtypeStruct((N, 128), jnp.float32)],
    )(d0, d1, h)


def _z_stats(s0, s1, u1, dinv, b1):
    BR = 1000
    G = N // BR

    def body(s0_ref, s1_ref, u_ref, d_ref, b_ref, z_ref, sum_ref, ssq_ref):
        z = d_ref[...][:, 0:1] * (s0_ref[...] + s1_ref[...] + u_ref[...]) + b_ref[...]
        z_ref[...] = z
        sum_ref[...] = jnp.sum(z, axis=0).reshape(1, 1, 128)
        ssq_ref[...] = jnp.sum(z * z, axis=0).reshape(1, 1, 128)

    return pl.pallas_call(
        body,
        grid=(G,),
        in_specs=[pl.BlockSpec((BR, 128), lambda i: (i, 0)),
                  pl.BlockSpec((BR, 128), lambda i: (i, 0)),
                  pl.BlockSpec((BR, 128), lambda i: (i, 0)),
                  pl.BlockSpec((BR, 8), lambda i: (i, 0)),
                  pl.BlockSpec((1, 128), lambda i: (0, 0))],
        out_specs=[pl.BlockSpec((BR, 128), lambda i: (i, 0)),
                   pl.BlockSpec((1, 1, 128), lambda i: (i, 0, 0)),
                   pl.BlockSpec((1, 1, 128), lambda i: (i, 0, 0))],
        out_shape=[jax.ShapeDtypeStruct((N, 128), jnp.float32),
                   jax.ShapeDtypeStruct((G, 1, 128), jnp.float32),
                   jax.ShapeDtypeStruct((G, 1, 128), jnp.float32)],
    )(s0, s1, u1, dinv, b1)


def _bn_relu_mm2(z, sums, ssq, gamma, beta, dinv, w2):
    BR = 1000
    G = N // BR

    def body(z_ref, sum_ref, ssq_ref, g_ref, b_ref, d_ref, w_ref, u2_ref):
        mean = jnp.sum(sum_ref[...], axis=0, keepdims=True) * (1.0 / N)
        var = jnp.sum(ssq_ref[...], axis=0, keepdims=True) * (1.0 / N) - mean * mean
        zb = (z_ref[...] - mean) * lax.rsqrt(var + 1e-5) * g_ref[...] + b_ref[...]
        h1 = jnp.maximum(zb, 0.0)
        u2_ref[...] = d_ref[...][:, 0:1] * lax.dot_general(
            h1, w_ref[...], (((1,), (1,)), ((), ())),
            preferred_element_type=jnp.float32)


    return pl.pallas_call(
        body,
        grid=(G,),
        in_specs=[pl.BlockSpec((BR, 128), lambda i: (i, 0)),
                  pl.BlockSpec((G, 128), lambda i: (0, 0)),
                  pl.BlockSpec((G, 128), lambda i: (0, 0)),
                  pl.BlockSpec((1, 128), lambda i: (0, 0)),
                  pl.BlockSpec((1, 128), lambda i: (0, 0)),
                  pl.BlockSpec((BR, 8), lambda i: (i, 0)),
                  pl.BlockSpec((128, 128), lambda i: (0, 0))],
        out_specs=pl.BlockSpec((BR, 128), lambda i: (i, 0)),
        out_shape=jax.ShapeDtypeStruct((N, 128), jnp.float32),
    )(z, sums, ssq, gamma, beta, dinv, w2)


def _final(s0, s1, u2, dinv, b2):
    BR = 1000

    def body(s0_ref, s1_ref, u_ref, d_ref, b_ref, o_ref):
        o_ref[...] = d_ref[...][:, 0:1] * (
            s0_ref[...] + s1_ref[...] + u_ref[...]) + b_ref[...]

    return pl.pallas_call(
        body,
        grid=(N // BR,),
        in_specs=[pl.BlockSpec((BR, 64), lambda i: (i, 0)),
                  pl.BlockSpec((BR, 64), lambda i: (i, 0)),
                  pl.BlockSpec((BR, 64), lambda i: (i, 0)),
                  pl.BlockSpec((BR, 8), lambda i: (i, 0)),
                  pl.BlockSpec((1, 64), lambda i: (0, 0))],
        out_specs=pl.BlockSpec((BR, 64), lambda i: (i, 0)),
        out_shape=jax.ShapeDtypeStruct((N, 64), jnp.float32),
    )(s0, s1, u2, dinv, b2)


def kernel(x, edge_index, W1, b1, gamma1, beta1, W2, b2):
    ei = edge_index.astype(jnp.int32)
    src, dst = ei[0], ei[1]
    pad = NW * EPW - E
    srcp = jnp.concatenate(
        [src, jnp.zeros((pad,), jnp.int32)]).reshape(NW, CH, K)
    dstp = jnp.concatenate(
        [dst, jnp.full((pad,), N, jnp.int32)]).reshape(NW, CH, K)

    degp = _make_sc_deg()(dstp)
    h = _matmul1(x, W1)
    dinv, u1 = _dinv_u1(degp[0, :N, :8], degp[1, :N, :8], h)
    s1p = _make_sc_agg(128)(u1, srcp, dstp)
    z, sums, ssq = _z_stats(s1p[0, :N], s1p[1, :N], u1, dinv,
                            b1.reshape(1, 128))
    sums = sums.reshape(-1, 128)
    ssq = ssq.reshape(-1, 128)
    w2p = jnp.concatenate([W2, jnp.zeros((64, 128), jnp.float32)], axis=0)
    u2 = _bn_relu_mm2(z, sums, ssq, gamma1.reshape(1, 128),
                      beta1.reshape(1, 128), dinv, w2p)
    s2p = _make_sc_agg(128)(u2, srcp, dstp)
    out = _final(s2p[0, :N, :64], s2p[1, :N, :64], u2[:, :64], dinv,
                 b2.reshape(1, 64))
    return out


# speedup vs baseline: 9.2105x; 1.1011x over previous
import functools

import jax
import jax.numpy as jnp
from jax import lax
from jax.experimental import pallas as pl
from jax.experimental.pallas import tpu as pltpu
from jax.experimental.pallas import tpu_sc as plsc

N = 10000
E = 320000
NPAD = 10240
NW = 32
EPW = 10240
K = 128
CH = EPW // K
RPT = NPAD // 16


def _zero_vmem(buf, rows, d):
    def zrow(i, _):
        def zcol(j, _):
            buf[i, pl.ds(j * 16, 16)] = jnp.zeros((16,), jnp.float32)
            return 0
        return lax.fori_loop(0, d // 16, zcol, 0)
    lax.fori_loop(0, rows, zrow, 0)


def _make_sc_deg():
    mesh = plsc.VectorSubcoreMesh(core_axis_name="c", subcore_axis_name="s")

    @functools.partial(
        pl.kernel, mesh=mesh,
        out_type=jax.ShapeDtypeStruct((2, NPAD, 128), jnp.float32),
        scratch_types=[
            pltpu.VMEM((CH, K), jnp.int32),
            pltpu.VMEM((K, 128), jnp.float32),
            pltpu.VMEM((K, 128), jnp.float32),
            pltpu.VMEM_SHARED((NPAD, 128), jnp.float32),
        ],
    )
    def deg_kernel(dst_hbm, out_hbm, dst_v, ones_v, zero_v, acc):
        cid = lax.axis_index("c")
        sid = lax.axis_index("s")
        wid = cid * 16 + sid
        pltpu.sync_copy(dst_hbm.at[wid], dst_v)

        def orow(i, _):
            def ocol(j, _):
                ones_v[i, pl.ds(j * 16, 16)] = jnp.ones((16,), jnp.float32)
                zero_v[i, pl.ds(j * 16, 16)] = jnp.zeros((16,), jnp.float32)
                return 0
            return lax.fori_loop(0, 8, ocol, 0)
        lax.fori_loop(0, K, orow, 0)
        base = sid * RPT
        for r in range(RPT // K):
            pltpu.sync_copy(zero_v, acc.at[pl.ds(base + r * K, K)])
        plsc.subcore_barrier()

        def chunk(c, _):
            pltpu.sync_copy(ones_v, acc.at[dst_v.at[c]], add=True)
            return 0
        lax.fori_loop(0, CH, chunk, 0)
        plsc.subcore_barrier()
        for r in range(RPT // K):
            sl = pl.ds(base + r * K, K)
            pltpu.sync_copy(acc.at[sl], out_hbm.at[cid, sl])

    return deg_kernel


def _make_sc_agg(d):
    mesh = plsc.VectorSubcoreMesh(core_axis_name="c", subcore_axis_name="s")

    NBUF = 2

    @functools.partial(
        pl.kernel, mesh=mesh,
        out_type=jax.ShapeDtypeStruct((2, NPAD, d), jnp.float32),
        scratch_types=[
            pltpu.VMEM((2, NBUF, K), jnp.int32),
            pltpu.VMEM((2, NBUF, K), jnp.int32),
            pltpu.VMEM_SHARED((NPAD, d), jnp.float32),
            pltpu.SemaphoreType.DMA,
        ] + [pltpu.VMEM((K, d), jnp.float32) for _ in range(NBUF)]
          + [pltpu.SemaphoreType.DMA for _ in range(NBUF)],
    )
    def agg_kernel(u_hbm, src_hbm, dst_hbm, out_hbm, src_v, dst_v, acc,
                   isem, *bufs_sems):
        rows = bufs_sems[:NBUF]
        sems = bufs_sems[NBUF:]
        cid = lax.axis_index("c")
        sid = lax.axis_index("s")
        wid = cid * 16 + sid
        _zero_vmem(rows[0], K, d)
        base = sid * RPT
        for r in range(RPT // K):
            pltpu.sync_copy(rows[0], acc.at[pl.ds(base + r * K, K)])
        plsc.subcore_barrier()

        def fetch_idx(p, slot):
            pltpu.async_copy(
                src_hbm.at[wid, pl.ds(p * NBUF, NBUF)], src_v.at[slot], isem)
            pltpu.async_copy(
                dst_hbm.at[wid, pl.ds(p * NBUF, NBUF)], dst_v.at[slot], isem)

        def wait_idx(p, slot):
            pltpu.make_async_copy(
                src_hbm.at[wid, pl.ds(p * NBUF, NBUF)], src_v.at[slot],
                isem).wait()
            pltpu.make_async_copy(
                dst_hbm.at[wid, pl.ds(p * NBUF, NBUF)], dst_v.at[slot],
                isem).wait()

        NG = CH // NBUF
        fetch_idx(0, 0)
        wait_idx(0, 0)
        fetch_idx(1, 1)
        for b in range(NBUF):
            pltpu.async_copy(u_hbm.at[src_v.at[0, b]], rows[b], sems[b])

        def group(p, _):
            slot = lax.rem(p, 2)
            nslot = lax.rem(p + 1, 2)
            for b in range(NBUF):
                pltpu.make_async_copy(u_hbm.at[src_v.at[slot, b]], rows[b],
                                      sems[b]).wait()
                pltpu.sync_copy(rows[b], acc.at[dst_v.at[slot, b]], add=True)
                if b == 0:
                    @pl.when(p + 1 < NG)
                    def _():
                        wait_idx(p + 1, nslot)

                @pl.when(p + 1 < NG)
                def _():
                    pltpu.async_copy(u_hbm.at[src_v.at[nslot, b]], rows[b],
                                     sems[b])

            @pl.when(p + 2 < NG)
            def _():
                fetch_idx(p + 2, slot)
            return 0
        lax.fori_loop(0, NG, group, 0)
        plsc.subcore_barrier()
        for r in range(RPT // K):
            sl = pl.ds(base + r * K, K)
            pltpu.sync_copy(acc.at[sl], out_hbm.at[cid, sl])

    return agg_kernel


def _matmul1(x, w1):
    BR = 1000

    def body(x_ref, w_ref, o_ref):
        o_ref[...] = lax.dot_general(
            x_ref[...], w_ref[...], (((1,), (1,)), ((), ())),
            preferred_element_type=jnp.float32)

    return pl.pallas_call(
        body,
        grid=(N // BR,),
        in_specs=[pl.BlockSpec((BR, 128), lambda i: (i, 0)),
                  pl.BlockSpec((128, 128), lambda i: (0, 0))],
        out_specs=pl.BlockSpec((BR, 128), lambda i: (i, 0)),
        out_shape=jax.ShapeDtypeStruct((N, 128), jnp.float32),
    )(x, w1)


def _dinv_u1(d0, d1, h):
    BR = 1000

    def body(d0_ref, d1_ref, h_ref, dinv_ref, u_ref):
        deg = d0_ref[...] + d1_ref[...] + 1.0
        dinv = lax.rsqrt(deg)
        dinv_ref[...] = dinv
        u_ref[...] = h_ref[...] * dinv[:, 0:1]

    return pl.pallas_call(
        body,
        grid=(N // BR,),
        in_specs=[pl.BlockSpec((BR, 8), lambda i: (i, 0)),
                  pl.BlockSpec((BR, 8), lambda i: (i, 0)),
                  pl.BlockSpec((BR, 128), lambda i: (i, 0))],
        out_specs=[pl.BlockSpec((BR, 8), lambda i: (i, 0)),
                   pl.BlockSpec((BR, 128), lambda i: (i, 0))],
        out_shape=[jax.ShapeDtypeStruct((N, 8), jnp.float32),
                   jax.ShapeDtypeStruct((N, 128), jnp.float32)],
    )(d0, d1, h)


def _z_stats(s0, s1, u1, dinv, b1):
    BR = 1000
    G = N // BR

    def body(s0_ref, s1_ref, u_ref, d_ref, b_ref, z_ref, sum_ref, ssq_ref):
        z = d_ref[...][:, 0:1] * (s0_ref[...] + s1_ref[...] + u_ref[...]) + b_ref[...]
        z_ref[...] = z
        sum_ref[...] = jnp.sum(z, axis=0).reshape(1, 1, 128)
        ssq_ref[...] = jnp.sum(z * z, axis=0).reshape(1, 1, 128)

    return pl.pallas_call(
        body,
        grid=(G,),
        in_specs=[pl.BlockSpec((BR, 128), lambda i: (i, 0)),
                  pl.BlockSpec((BR, 128), lambda i: (i, 0)),
                  pl.BlockSpec((BR, 128), lambda i: (i, 0)),
                  pl.BlockSpec((BR, 8), lambda i: (i, 0)),
                  pl.BlockSpec((1, 128), lambda i: (0, 0))],
        out_specs=[pl.BlockSpec((BR, 128), lambda i: (i, 0)),
                   pl.BlockSpec((1, 1, 128), lambda i: (i, 0, 0)),
                   pl.BlockSpec((1, 1, 128), lambda i: (i, 0, 0))],
        out_shape=[jax.ShapeDtypeStruct((N, 128), jnp.float32),
                   jax.ShapeDtypeStruct((G, 1, 128), jnp.float32),
                   jax.ShapeDtypeStruct((G, 1, 128), jnp.float32)],
    )(s0, s1, u1, dinv, b1)


def _bn_relu_mm2(z, sums, ssq, gamma, beta, dinv, w2):
    BR = 1000
    G = N // BR

    def body(z_ref, sum_ref, ssq_ref, g_ref, b_ref, d_ref, w_ref, u2_ref):
        mean = jnp.sum(sum_ref[...], axis=0, keepdims=True) * (1.0 / N)
        var = jnp.sum(ssq_ref[...], axis=0, keepdims=True) * (1.0 / N) - mean * mean
        zb = (z_ref[...] - mean) * lax.rsqrt(var + 1e-5) * g_ref[...] + b_ref[...]
        h1 = jnp.maximum(zb, 0.0)
        u2_ref[...] = d_ref[...][:, 0:1] * lax.dot_general(
            h1, w_ref[...], (((1,), (1,)), ((), ())),
            preferred_element_type=jnp.float32)


    return pl.pallas_call(
        body,
        grid=(G,),
        in_specs=[pl.BlockSpec((BR, 128), lambda i: (i, 0)),
                  pl.BlockSpec((G, 128), lambda i: (0, 0)),
                  pl.BlockSpec((G, 128), lambda i: (0, 0)),
                  pl.BlockSpec((1, 128), lambda i: (0, 0)),
                  pl.BlockSpec((1, 128), lambda i: (0, 0)),
                  pl.BlockSpec((BR, 8), lambda i: (i, 0)),
                  pl.BlockSpec((128, 128), lambda i: (0, 0))],
        out_specs=pl.BlockSpec((BR, 128), lambda i: (i, 0)),
        out_shape=jax.ShapeDtypeStruct((N, 128), jnp.float32),
    )(z, sums, ssq, gamma, beta, dinv, w2)


def _final(s0, s1, u2, dinv, b2):
    BR = 1000

    def body(s0_ref, s1_ref, u_ref, d_ref, b_ref, o_ref):
        o_ref[...] = d_ref[...][:, 0:1] * (
            s0_ref[...] + s1_ref[...] + u_ref[...]) + b_ref[...]

    return pl.pallas_call(
        body,
        grid=(N // BR,),
        in_specs=[pl.BlockSpec((BR, 64), lambda i: (i, 0)),
                  pl.BlockSpec((BR, 64), lambda i: (i, 0)),
                  pl.BlockSpec((BR, 64), lambda i: (i, 0)),
                  pl.BlockSpec((BR, 8), lambda i: (i, 0)),
                  pl.BlockSpec((1, 64), lambda i: (0, 0))],
        out_specs=pl.BlockSpec((BR, 64), lambda i: (i, 0)),
        out_shape=jax.ShapeDtypeStruct((N, 64), jnp.float32),
    )(s0, s1, u2, dinv, b2)


def kernel(x, edge_index, W1, b1, gamma1, beta1, W2, b2):
    ei = edge_index.astype(jnp.int32)
    src, dst = ei[0], ei[1]
    pad = NW * EPW - E
    srcp = jnp.concatenate(
        [src, jnp.zeros((pad,), jnp.int32)]).reshape(NW, CH, K)
    dstp = jnp.concatenate(
        [dst, jnp.full((pad,), N, jnp.int32)]).reshape(NW, CH, K)

    degp = _make_sc_deg()(dstp)
    h = _matmul1(x, W1)
    dinv, u1 = _dinv_u1(degp[0, :N, :8], degp[1, :N, :8], h)
    s1p = _make_sc_agg(128)(u1, srcp, dstp)
    z, sums, ssq = _z_stats(s1p[0, :N], s1p[1, :N], u1, dinv,
                            b1.reshape(1, 128))
    sums = sums.reshape(-1, 128)
    ssq = ssq.reshape(-1, 128)
    w2p = jnp.concatenate([W2, jnp.zeros((64, 128), jnp.float32)], axis=0)
    u2 = _bn_relu_mm2(z, sums, ssq, gamma1.reshape(1, 128),
                      beta1.reshape(1, 128), dinv, w2p)
    s2p = _make_sc_agg(128)(u2, srcp, dstp)
    out = _final(s2p[0, :N, :64], s2p[1, :N, :64], u2[:, :64], dinv,
                 b2.reshape(1, 64))
    return out


# speedup vs baseline: 9.7336x; 1.0568x over previous
import functools

import jax
import jax.numpy as jnp
from jax import lax
from jax.experimental import pallas as pl
from jax.experimental.pallas import tpu as pltpu
from jax.experimental.pallas import tpu_sc as plsc

N = 10000
E = 320000
NPAD = 10240
NW = 32
K = 128
NCH = 2560
CH = NCH // NW
RPT = NPAD // 16


def _zero_vmem(buf, rows, d):
    def zrow(i, _):
        def zcol(j, _):
            buf[i, pl.ds(j * 16, 16)] = jnp.zeros((16,), jnp.float32)
            return 0
        return lax.fori_loop(0, d // 16, zcol, 0)
    lax.fori_loop(0, rows, zrow, 0)


def _make_sc_deg():
    mesh = plsc.VectorSubcoreMesh(core_axis_name="c", subcore_axis_name="s")

    @functools.partial(
        pl.kernel, mesh=mesh,
        out_type=jax.ShapeDtypeStruct((2, NPAD, 128), jnp.float32),
        scratch_types=[
            pltpu.VMEM((CH, K), jnp.int32),
            pltpu.VMEM((K, 128), jnp.float32),
            pltpu.VMEM((K, 128), jnp.float32),
            pltpu.VMEM_SHARED((NPAD, 128), jnp.float32),
        ],
    )
    def deg_kernel(dst_hbm, out_hbm, dst_v, ones_v, zero_v, acc):
        cid = lax.axis_index("c")
        sid = lax.axis_index("s")
        wid = cid * 16 + sid
        pltpu.sync_copy(dst_hbm.at[pl.ds(wid * CH, CH)], dst_v)

        def orow(i, _):
            def ocol(j, _):
                ones_v[i, pl.ds(j * 16, 16)] = jnp.ones((16,), jnp.float32)
                zero_v[i, pl.ds(j * 16, 16)] = jnp.zeros((16,), jnp.float32)
                return 0
            return lax.fori_loop(0, 8, ocol, 0)
        lax.fori_loop(0, K, orow, 0)
        base = sid * RPT
        for r in range(RPT // K):
            pltpu.sync_copy(zero_v, acc.at[pl.ds(base + r * K, K)])
        plsc.subcore_barrier()

        def chunk(c, _):
            pltpu.sync_copy(ones_v, acc.at[dst_v.at[c]], add=True)
            return 0
        lax.fori_loop(0, CH, chunk, 0)
        plsc.subcore_barrier()
        for r in range(RPT // K):
            sl = pl.ds(base + r * K, K)
            pltpu.sync_copy(acc.at[sl], out_hbm.at[cid, sl])

    return deg_kernel


def _make_sc_agg(d, ch0, ch1):
    mesh = plsc.VectorSubcoreMesh(core_axis_name="c", subcore_axis_name="s")

    NBUF = 2
    assert 16 * (ch0 + ch1) == NCH
    assert ch0 % NBUF == 0 and ch1 % NBUF == 0 and ch0 >= 2 * NBUF

    @functools.partial(
        pl.kernel, mesh=mesh,
        out_type=jax.ShapeDtypeStruct((2, NPAD, d), jnp.float32),
        scratch_types=[
            pltpu.VMEM((2, NBUF, K), jnp.int32),
            pltpu.VMEM((2, NBUF, K), jnp.int32),
            pltpu.VMEM_SHARED((NPAD, d), jnp.float32),
            pltpu.SemaphoreType.DMA,
        ] + [pltpu.VMEM((K, d), jnp.float32) for _ in range(NBUF)]
          + [pltpu.SemaphoreType.DMA for _ in range(NBUF)],
    )
    def agg_kernel(u_hbm, src_hbm, dst_hbm, out_hbm, src_v, dst_v, acc,
                   isem, *bufs_sems):
        rows = bufs_sems[:NBUF]
        sems = bufs_sems[NBUF:]
        cid = lax.axis_index("c")
        sid = lax.axis_index("s")
        cbase = jnp.where(cid == 0, sid * ch0, 16 * ch0 + sid * ch1)
        nch_me = jnp.where(cid == 0, ch0, ch1)
        _zero_vmem(rows[0], K, d)
        base = sid * RPT
        for r in range(RPT // K):
            pltpu.sync_copy(rows[0], acc.at[pl.ds(base + r * K, K)])
        plsc.subcore_barrier()

        def fetch_idx(p, slot):
            pltpu.async_copy(
                src_hbm.at[pl.ds(cbase + p * NBUF, NBUF)], src_v.at[slot],
                isem)
            pltpu.async_copy(
                dst_hbm.at[pl.ds(cbase + p * NBUF, NBUF)], dst_v.at[slot],
                isem)

        def wait_idx(p, slot):
            pltpu.make_async_copy(
                src_hbm.at[pl.ds(cbase + p * NBUF, NBUF)], src_v.at[slot],
                isem).wait()
            pltpu.make_async_copy(
                dst_hbm.at[pl.ds(cbase + p * NBUF, NBUF)], dst_v.at[slot],
                isem).wait()

        NG = nch_me // NBUF
        fetch_idx(0, 0)
        wait_idx(0, 0)
        fetch_idx(1, 1)
        for b in range(NBUF):
            pltpu.async_copy(u_hbm.at[src_v.at[0, b]], rows[b], sems[b])

        def group(p, _):
            slot = lax.rem(p, 2)
            nslot = lax.rem(p + 1, 2)
            for b in range(NBUF):
                pltpu.make_async_copy(u_hbm.at[src_v.at[slot, b]], rows[b],
                                      sems[b]).wait()
                pltpu.sync_copy(rows[b], acc.at[dst_v.at[slot, b]], add=True)
                if b == 0:
                    @pl.when(p + 1 < NG)
                    def _():
                        wait_idx(p + 1, nslot)

                @pl.when(p + 1 < NG)
                def _():
                    pltpu.async_copy(u_hbm.at[src_v.at[nslot, b]], rows[b],
                                     sems[b])

            @pl.when(p + 2 < NG)
            def _():
                fetch_idx(p + 2, slot)
            return 0
        lax.fori_loop(0, NG, group, 0)
        plsc.subcore_barrier()
        for r in range(RPT // K):
            sl = pl.ds(base + r * K, K)
            pltpu.sync_copy(acc.at[sl], out_hbm.at[cid, sl])

    return agg_kernel


def _matmul1(x, w1):
    BR = 1000

    def body(x_ref, w_ref, o_ref):
        o_ref[...] = lax.dot_general(
            x_ref[...], w_ref[...], (((1,), (1,)), ((), ())),
            preferred_element_type=jnp.float32)

    return pl.pallas_call(
        body,
        grid=(N // BR,),
        in_specs=[pl.BlockSpec((BR, 128), lambda i: (i, 0)),
                  pl.BlockSpec((128, 128), lambda i: (0, 0))],
        out_specs=pl.BlockSpec((BR, 128), lambda i: (i, 0)),
        out_shape=jax.ShapeDtypeStruct((N, 128), jnp.float32),
    )(x, w1)


def _dinv_u1(d0, d1, h):
    BR = 1000

    def body(d0_ref, d1_ref, h_ref, dinv_ref, u_ref):
        deg = d0_ref[...] + d1_ref[...] + 1.0
        dinv = lax.rsqrt(deg)
        dinv_ref[...] = dinv
        u_ref[...] = h_ref[...] * dinv[:, 0:1]

    return pl.pallas_call(
        body,
        grid=(N // BR,),
        in_specs=[pl.BlockSpec((BR, 8), lambda i: (i, 0)),
                  pl.BlockSpec((BR, 8), lambda i: (i, 0)),
                  pl.BlockSpec((BR, 128), lambda i: (i, 0))],
        out_specs=[pl.BlockSpec((BR, 8), lambda i: (i, 0)),
                   pl.BlockSpec((BR, 128), lambda i: (i, 0))],
        out_shape=[jax.ShapeDtypeStruct((N, 8), jnp.float32),
                   jax.ShapeDtypeStruct((N, 128), jnp.float32)],
    )(d0, d1, h)


def _z_stats(s0, s1, u1, dinv, b1):
    BR = 1000
    G = N // BR

    def body(s0_ref, s1_ref, u_ref, d_ref, b_ref, z_ref, sum_ref, ssq_ref):
        z = d_ref[...][:, 0:1] * (s0_ref[...] + s1_ref[...] + u_ref[...]) + b_ref[...]
        z_ref[...] = z
        sum_ref[...] = jnp.sum(z, axis=0).reshape(1, 1, 128)
        ssq_ref[...] = jnp.sum(z * z, axis=0).reshape(1, 1, 128)

    return pl.pallas_call(
        body,
        grid=(G,),
        in_specs=[pl.BlockSpec((BR, 128), lambda i: (i, 0)),
                  pl.BlockSpec((BR, 128), lambda i: (i, 0)),
                  pl.BlockSpec((BR, 128), lambda i: (i, 0)),
                  pl.BlockSpec((BR, 8), lambda i: (i, 0)),
                  pl.BlockSpec((1, 128), lambda i: (0, 0))],
        out_specs=[pl.BlockSpec((BR, 128), lambda i: (i, 0)),
                   pl.BlockSpec((1, 1, 128), lambda i: (i, 0, 0)),
                   pl.BlockSpec((1, 1, 128), lambda i: (i, 0, 0))],
        out_shape=[jax.ShapeDtypeStruct((N, 128), jnp.float32),
                   jax.ShapeDtypeStruct((G, 1, 128), jnp.float32),
                   jax.ShapeDtypeStruct((G, 1, 128), jnp.float32)],
    )(s0, s1, u1, dinv, b1)


def _bn_relu_mm2(z, sums, ssq, gamma, beta, dinv, w2):
    BR = 1000
    G = N // BR

    def body(z_ref, sum_ref, ssq_ref, g_ref, b_ref, d_ref, w_ref, u2_ref):
        mean = jnp.sum(sum_ref[...], axis=0, keepdims=True) * (1.0 / N)
        var = jnp.sum(ssq_ref[...], axis=0, keepdims=True) * (1.0 / N) - mean * mean
        zb = (z_ref[...] - mean) * lax.rsqrt(var + 1e-5) * g_ref[...] + b_ref[...]
        h1 = jnp.maximum(zb, 0.0)
        u2_ref[...] = d_ref[...][:, 0:1] * lax.dot_general(
            h1, w_ref[...], (((1,), (1,)), ((), ())),
            preferred_element_type=jnp.float32)


    return pl.pallas_call(
        body,
        grid=(G,),
        in_specs=[pl.BlockSpec((BR, 128), lambda i: (i, 0)),
                  pl.BlockSpec((G, 128), lambda i: (0, 0)),
                  pl.BlockSpec((G, 128), lambda i: (0, 0)),
                  pl.BlockSpec((1, 128), lambda i: (0, 0)),
                  pl.BlockSpec((1, 128), lambda i: (0, 0)),
                  pl.BlockSpec((BR, 8), lambda i: (i, 0)),
                  pl.BlockSpec((128, 128), lambda i: (0, 0))],
        out_specs=pl.BlockSpec((BR, 128), lambda i: (i, 0)),
        out_shape=jax.ShapeDtypeStruct((N, 128), jnp.float32),
    )(z, sums, ssq, gamma, beta, dinv, w2)


def _final(s0, s1, u2, dinv, b2):
    BR = 1000

    def body(s0_ref, s1_ref, u_ref, d_ref, b_ref, o_ref):
        o_ref[...] = d_ref[...][:, 0:1] * (
            s0_ref[...] + s1_ref[...] + u_ref[...]) + b_ref[...]

    return pl.pallas_call(
        body,
        grid=(N // BR,),
        in_specs=[pl.BlockSpec((BR, 64), lambda i: (i, 0)),
                  pl.BlockSpec((BR, 64), lambda i: (i, 0)),
                  pl.BlockSpec((BR, 64), lambda i: (i, 0)),
                  pl.BlockSpec((BR, 8), lambda i: (i, 0)),
                  pl.BlockSpec((1, 64), lambda i: (0, 0))],
        out_specs=pl.BlockSpec((BR, 64), lambda i: (i, 0)),
        out_shape=jax.ShapeDtypeStruct((N, 64), jnp.float32),
    )(s0, s1, u2, dinv, b2)


def kernel(x, edge_index, W1, b1, gamma1, beta1, W2, b2):
    ei = edge_index.astype(jnp.int32)
    src, dst = ei[0], ei[1]
    pad = NCH * K - E
    srcp = jnp.concatenate(
        [src, jnp.zeros((pad,), jnp.int32)]).reshape(NCH, K)
    dstp = jnp.concatenate(
        [dst, jnp.full((pad,), N, jnp.int32)]).reshape(NCH, K)

    CH0, CH1 = 32, 128
    degp = _make_sc_deg()(dstp)
    h = _matmul1(x, W1)
    dinv, u1 = _dinv_u1(degp[0, :N, :8], degp[1, :N, :8], h)
    s1p = _make_sc_agg(128, CH0, CH1)(u1, srcp, dstp)
    z, sums, ssq = _z_stats(s1p[0, :N], s1p[1, :N], u1, dinv,
                            b1.reshape(1, 128))
    sums = sums.reshape(-1, 128)
    ssq = ssq.reshape(-1, 128)
    w2p = jnp.concatenate([W2, jnp.zeros((64, 128), jnp.float32)], axis=0)
    u2 = _bn_relu_mm2(z, sums, ssq, gamma1.reshape(1, 128),
                      beta1.reshape(1, 128), dinv, w2p)
    s2p = _make_sc_agg(128, CH0, CH1)(u2, srcp, dstp)
    out = _final(s2p[0, :N, :64], s2p[1, :N, :64], u2[:, :64], dinv,
                 b2.reshape(1, 64))
    return out


# speedup vs baseline: 23.5144x; 2.4158x over previous
import functools

import jax
import jax.numpy as jnp
from jax import lax
from jax.experimental import pallas as pl
from jax.experimental.pallas import tpu as pltpu
from jax.experimental.pallas import tpu_sc as plsc

N = 10000
E = 320000
NPAD = 10240
NW = 32
K = 128
NCH = 2560
CH = NCH // NW
RPT = NPAD // 16


def _zero_vmem(buf, rows, d):
    def zrow(i, _):
        def zcol(j, _):
            buf[i, pl.ds(j * 16, 16)] = jnp.zeros((16,), jnp.float32)
            return 0
        return lax.fori_loop(0, d // 16, zcol, 0)
    lax.fori_loop(0, rows, zrow, 0)


def _make_sc_deg():
    mesh = plsc.VectorSubcoreMesh(core_axis_name="c", subcore_axis_name="s")

    @functools.partial(
        pl.kernel, mesh=mesh,
        out_type=jax.ShapeDtypeStruct((2, NPAD, 128), jnp.float32),
        scratch_types=[
            pltpu.VMEM((CH, K), jnp.int32),
            pltpu.VMEM((K, 128), jnp.float32),
            pltpu.VMEM((K, 128), jnp.float32),
            pltpu.VMEM_SHARED((NPAD, 128), jnp.float32),
        ],
    )
    def deg_kernel(dst_hbm, out_hbm, dst_v, ones_v, zero_v, acc):
        cid = lax.axis_index("c")
        sid = lax.axis_index("s")
        wid = cid * 16 + sid
        pltpu.sync_copy(dst_hbm.at[pl.ds(wid * CH, CH)], dst_v)

        def orow(i, _):
            def ocol(j, _):
                ones_v[i, pl.ds(j * 16, 16)] = jnp.ones((16,), jnp.float32)
                zero_v[i, pl.ds(j * 16, 16)] = jnp.zeros((16,), jnp.float32)
                return 0
            return lax.fori_loop(0, 8, ocol, 0)
        lax.fori_loop(0, K, orow, 0)
        base = sid * RPT
        for r in range(RPT // K):
            pltpu.sync_copy(zero_v, acc.at[pl.ds(base + r * K, K)])
        plsc.subcore_barrier()

        def chunk(c, _):
            pltpu.sync_copy(ones_v, acc.at[dst_v.at[c]], add=True)
            return 0
        lax.fori_loop(0, CH, chunk, 0)
        plsc.subcore_barrier()
        for r in range(RPT // K):
            sl = pl.ds(base + r * K, K)
            pltpu.sync_copy(acc.at[sl], out_hbm.at[cid, sl])

    return deg_kernel


def _make_sc_agg(split_edges):
    mesh = plsc.VectorSubcoreMesh(core_axis_name="c", subcore_axis_name="s")

    W = 64
    NBUF = 2
    CHPT = (NCH // NW) if split_edges else (NCH // 16)

    @functools.partial(
        pl.kernel, mesh=mesh,
        compiler_params=pltpu.CompilerParams(use_tc_tiling_on_sc=False),
        out_type=jax.ShapeDtypeStruct((2, NPAD, W), jnp.float32),
        scratch_types=[
            pltpu.VMEM((2, NBUF, K), jnp.int32),
            pltpu.VMEM((2, NBUF, K), jnp.int32),
            pltpu.VMEM_SHARED((NPAD, W), jnp.float32),
            pltpu.VMEM_SHARED((NPAD, W), jnp.float32),
            pltpu.SemaphoreType.DMA,
        ] + [pltpu.VMEM((K, W), jnp.float32) for _ in range(NBUF)]
          + [pltpu.SemaphoreType.DMA for _ in range(NBUF)],
    )
    def agg_kernel(u0_hbm, u1_hbm, src_hbm, dst_hbm, out_hbm, src_v, dst_v,
                   u_sp, acc, isem, *bufs_sems):
        rows = bufs_sems[:NBUF]
        sems = bufs_sems[NBUF:]
        cid = lax.axis_index("c")
        sid = lax.axis_index("s")
        base = sid * RPT
        sl_mine = pl.ds(base, RPT)

        @pl.when(cid == 0)
        def _():
            pltpu.sync_copy(u0_hbm.at[sl_mine], u_sp.at[sl_mine])

        @pl.when(cid == 1)
        def _():
            pltpu.sync_copy(u1_hbm.at[sl_mine], u_sp.at[sl_mine])

        cbase = (cid * (NCH // 2) + sid * CHPT) if split_edges else sid * CHPT
        _zero_vmem(rows[0], K, W)
        for r in range(RPT // K):
            pltpu.sync_copy(rows[0], acc.at[pl.ds(base + r * K, K)])
        plsc.subcore_barrier()

        def fetch_idx(p, slot):
            pltpu.async_copy(
                src_hbm.at[pl.ds(cbase + p * NBUF, NBUF)], src_v.at[slot],
                isem)
            pltpu.async_copy(
                dst_hbm.at[pl.ds(cbase + p * NBUF, NBUF)], dst_v.at[slot],
                isem)

        def wait_idx(p, slot):
            pltpu.make_async_copy(
                src_hbm.at[pl.ds(cbase + p * NBUF, NBUF)], src_v.at[slot],
                isem).wait()
            pltpu.make_async_copy(
                dst_hbm.at[pl.ds(cbase + p * NBUF, NBUF)], dst_v.at[slot],
                isem).wait()

        NG = CHPT // NBUF
        fetch_idx(0, 0)
        wait_idx(0, 0)
        fetch_idx(1, 1)
        for b in range(NBUF):
            pltpu.async_copy(u_sp.at[src_v.at[0, b]], rows[b], sems[b])

        def group(p, _):
            slot = lax.rem(p, 2)
            nslot = lax.rem(p + 1, 2)
            for b in range(NBUF):
                pltpu.make_async_copy(u_sp.at[src_v.at[slot, b]], rows[b],
                                      sems[b]).wait()
                pltpu.sync_copy(rows[b], acc.at[dst_v.at[slot, b]], add=True)
                if b == 0:
                    @pl.when(p + 1 < NG)
                    def _():
                        wait_idx(p + 1, nslot)

                @pl.when(p + 1 < NG)
                def _():
                    pltpu.async_copy(u_sp.at[src_v.at[nslot, b]], rows[b],
                                     sems[b])

            @pl.when(p + 2 < NG)
            def _():
                fetch_idx(p + 2, slot)
            return 0
        lax.fori_loop(0, NG, group, 0)
        plsc.subcore_barrier()
        for r in range(RPT // K):
            sl = pl.ds(base + r * K, K)
            pltpu.sync_copy(acc.at[sl], out_hbm.at[cid, sl])

    return agg_kernel


def _matmul1(x, w1):
    BR = 1000

    def body(x_ref, w_ref, o_ref):
        o_ref[...] = lax.dot_general(
            x_ref[...], w_ref[...], (((1,), (1,)), ((), ())),
            preferred_element_type=jnp.float32)

    return pl.pallas_call(
        body,
        grid=(N // BR,),
        in_specs=[pl.BlockSpec((BR, 128), lambda i: (i, 0)),
                  pl.BlockSpec((128, 128), lambda i: (0, 0))],
        out_specs=pl.BlockSpec((BR, 128), lambda i: (i, 0)),
        out_shape=jax.ShapeDtypeStruct((N, 128), jnp.float32),
    )(x, w1)


def _dinv_u1(d0, d1, h):
    BR = 1000

    def body(d0_ref, d1_ref, h_ref, dinv_ref, u_ref):
        deg = d0_ref[...] + d1_ref[...] + 1.0
        dinv = lax.rsqrt(deg)
        dinv_ref[...] = dinv
        u_ref[...] = h_ref[...] * dinv[:, 0:1]

    return pl.pallas_call(
        body,
        grid=(N // BR,),
        in_specs=[pl.BlockSpec((BR, 8), lambda i: (i, 0)),
                  pl.BlockSpec((BR, 8), lambda i: (i, 0)),
                  pl.BlockSpec((BR, 128), lambda i: (i, 0))],
        out_specs=[pl.BlockSpec((BR, 8), lambda i: (i, 0)),
                   pl.BlockSpec((BR, 128), lambda i: (i, 0))],
        out_shape=[jax.ShapeDtypeStruct((N, 8), jnp.float32),
                   jax.ShapeDtypeStruct((N, 128), jnp.float32)],
    )(d0, d1, h)


def _z_stats(s0, s1, u1, dinv, b1):
    BR = 1000
    G = N // BR

    def body(s0_ref, s1_ref, u_ref, d_ref, b_ref, z_ref, sum_ref, ssq_ref):
        s = jnp.concatenate([s0_ref[...], s1_ref[...]], axis=1)
        z = d_ref[...][:, 0:1] * (s + u_ref[...]) + b_ref[...]
        z_ref[...] = z
        sum_ref[...] = jnp.sum(z, axis=0).reshape(1, 1, 128)
        ssq_ref[...] = jnp.sum(z * z, axis=0).reshape(1, 1, 128)

    return pl.pallas_call(
        body,
        grid=(G,),
        in_specs=[pl.BlockSpec((BR, 64), lambda i: (i, 0)),
                  pl.BlockSpec((BR, 64), lambda i: (i, 0)),
                  pl.BlockSpec((BR, 128), lambda i: (i, 0)),
                  pl.BlockSpec((BR, 8), lambda i: (i, 0)),
                  pl.BlockSpec((1, 128), lambda i: (0, 0))],
        out_specs=[pl.BlockSpec((BR, 128), lambda i: (i, 0)),
                   pl.BlockSpec((1, 1, 128), lambda i: (i, 0, 0)),
                   pl.BlockSpec((1, 1, 128), lambda i: (i, 0, 0))],
        out_shape=[jax.ShapeDtypeStruct((N, 128), jnp.float32),
                   jax.ShapeDtypeStruct((G, 1, 128), jnp.float32),
                   jax.ShapeDtypeStruct((G, 1, 128), jnp.float32)],
    )(s0, s1, u1, dinv, b1)


def _bn_relu_mm2(z, sums, ssq, gamma, beta, dinv, w2):
    BR = 1000
    G = N // BR

    def body(z_ref, sum_ref, ssq_ref, g_ref, b_ref, d_ref, w_ref, u2_ref):
        mean = jnp.sum(sum_ref[...], axis=0, keepdims=True) * (1.0 / N)
        var = jnp.sum(ssq_ref[...], axis=0, keepdims=True) * (1.0 / N) - mean * mean
        zb = (z_ref[...] - mean) * lax.rsqrt(var + 1e-5) * g_ref[...] + b_ref[...]
        h1 = jnp.maximum(zb, 0.0)
        u2_ref[...] = d_ref[...][:, 0:1] * lax.dot_general(
            h1, w_ref[...], (((1,), (1,)), ((), ())),
            preferred_element_type=jnp.float32)

    return pl.pallas_call(
        body,
        grid=(G,),
        in_specs=[pl.BlockSpec((BR, 128), lambda i: (i, 0)),
                  pl.BlockSpec((G, 128), lambda i: (0, 0)),
                  pl.BlockSpec((G, 128), lambda i: (0, 0)),
                  pl.BlockSpec((1, 128), lambda i: (0, 0)),
                  pl.BlockSpec((1, 128), lambda i: (0, 0)),
                  pl.BlockSpec((BR, 8), lambda i: (i, 0)),
                  pl.BlockSpec((64, 128), lambda i: (0, 0))],
        out_specs=pl.BlockSpec((BR, 64), lambda i: (i, 0)),
        out_shape=jax.ShapeDtypeStruct((N, 64), jnp.float32),
    )(z, sums, ssq, gamma, beta, dinv, w2)


def _final(s0, s1, u2, dinv, b2):
    BR = 1000

    def body(s0_ref, s1_ref, u_ref, d_ref, b_ref, o_ref):
        o_ref[...] = d_ref[...][:, 0:1] * (
            s0_ref[...] + s1_ref[...] + u_ref[...]) + b_ref[...]

    return pl.pallas_call(
        body,
        grid=(N // BR,),
        in_specs=[pl.BlockSpec((BR, 64), lambda i: (i, 0)),
                  pl.BlockSpec((BR, 64), lambda i: (i, 0)),
                  pl.BlockSpec((BR, 64), lambda i: (i, 0)),
                  pl.BlockSpec((BR, 8), lambda i: (i, 0)),
                  pl.BlockSpec((1, 64), lambda i: (0, 0))],
        out_specs=pl.BlockSpec((BR, 64), lambda i: (i, 0)),
        out_shape=jax.ShapeDtypeStruct((N, 64), jnp.float32),
    )(s0, s1, u2, dinv, b2)


def kernel(x, edge_index, W1, b1, gamma1, beta1, W2, b2):
    ei = edge_index.astype(jnp.int32)
    src, dst = ei[0], ei[1]
    pad = NCH * K - E
    srcp = jnp.concatenate(
        [src, jnp.zeros((pad,), jnp.int32)]).reshape(NCH, K)
    dstp = jnp.concatenate(
        [dst, jnp.full((pad,), N, jnp.int32)]).reshape(NCH, K)

    degp = _make_sc_deg()(dstp)
    h = _matmul1(x, W1)
    dinv, u1 = _dinv_u1(degp[0, :N, :8], degp[1, :N, :8], h)
    zpad = jnp.zeros((NPAD - N, 64), jnp.float32)
    u1p = jnp.concatenate([u1, jnp.zeros((NPAD - N, 128), jnp.float32)])
    s1p = _make_sc_agg(False)(u1p[:, :64], u1p[:, 64:], srcp, dstp)
    z, sums, ssq = _z_stats(s1p[0, :N], s1p[1, :N], u1, dinv,
                            b1.reshape(1, 128))
    sums = sums.reshape(-1, 128)
    ssq = ssq.reshape(-1, 128)
    u2 = _bn_relu_mm2(z, sums, ssq, gamma1.reshape(1, 128),
                      beta1.reshape(1, 128), dinv, W2)
    u2p = jnp.concatenate([u2, zpad])
    s2p = _make_sc_agg(True)(u2p, u2p, srcp, dstp)
    out = _final(s2p[0, :N], s2p[1, :N], u2, dinv, b2.reshape(1, 64))
    return out


# speedup vs baseline: 25.9320x; 1.1028x over previous
import functools

import jax
import jax.numpy as jnp
from jax import lax
from jax.experimental import pallas as pl
from jax.experimental.pallas import tpu as pltpu
from jax.experimental.pallas import tpu_sc as plsc

N = 10000
E = 320000
NPAD = 10240
NW = 32
K = 128
NCH = 2560
CH = NCH // NW
RPT = NPAD // 16


def _zero_vmem(buf, rows, d):
    def zrow(i, _):
        def zcol(j, _):
            buf[i, pl.ds(j * 16, 16)] = jnp.zeros((16,), jnp.float32)
            return 0
        return lax.fori_loop(0, d // 16, zcol, 0)
    lax.fori_loop(0, rows, zrow, 0)


def _make_sc_deg():
    mesh = plsc.VectorSubcoreMesh(core_axis_name="c", subcore_axis_name="s")

    @functools.partial(
        pl.kernel, mesh=mesh,
        compiler_params=pltpu.CompilerParams(use_tc_tiling_on_sc=False),
        out_type=jax.ShapeDtypeStruct((2, NPAD, 16), jnp.float32),
        scratch_types=[
            pltpu.VMEM((CH, K), jnp.int32),
            pltpu.VMEM((K, 16), jnp.float32),
            pltpu.VMEM((K, 16), jnp.float32),
            pltpu.VMEM_SHARED((NPAD, 16), jnp.float32),
        ],
    )
    def deg_kernel(dst_hbm, out_hbm, dst_v, ones_v, zero_v, acc):
        cid = lax.axis_index("c")
        sid = lax.axis_index("s")
        wid = cid * 16 + sid
        pltpu.sync_copy(dst_hbm.at[pl.ds(wid * CH, CH)], dst_v)

        def orow(i, _):
            ones_v[i, pl.ds(0, 16)] = jnp.ones((16,), jnp.float32)
            zero_v[i, pl.ds(0, 16)] = jnp.zeros((16,), jnp.float32)
            return 0
        lax.fori_loop(0, K, orow, 0)
        base = sid * RPT
        for r in range(RPT // K):
            pltpu.sync_copy(zero_v, acc.at[pl.ds(base + r * K, K)])
        plsc.subcore_barrier()

        def chunk(c, _):
            pltpu.sync_copy(ones_v, acc.at[dst_v.at[c]], add=True)
            return 0
        lax.fori_loop(0, CH, chunk, 0)
        plsc.subcore_barrier()
        for r in range(RPT // K):
            sl = pl.ds(base + r * K, K)
            pltpu.sync_copy(acc.at[sl], out_hbm.at[cid, sl])

    return deg_kernel


def _make_sc_agg(split_edges):
    mesh = plsc.VectorSubcoreMesh(core_axis_name="c", subcore_axis_name="s")

    W = 64
    NBUF = 2
    CHPT = (NCH // NW) if split_edges else (NCH // 16)

    @functools.partial(
        pl.kernel, mesh=mesh,
        compiler_params=pltpu.CompilerParams(use_tc_tiling_on_sc=False),
        out_type=jax.ShapeDtypeStruct((2, NPAD, W), jnp.float32),
        scratch_types=[
            pltpu.VMEM((2, NBUF, K), jnp.int32),
            pltpu.VMEM((2, NBUF, K), jnp.int32),
            pltpu.VMEM_SHARED((NPAD, W), jnp.float32),
            pltpu.VMEM_SHARED((NPAD, W), jnp.float32),
            pltpu.SemaphoreType.DMA,
        ] + [pltpu.VMEM((K, W), jnp.float32) for _ in range(NBUF)]
          + [pltpu.SemaphoreType.DMA for _ in range(NBUF)],
    )
    def agg_kernel(u0_hbm, u1_hbm, src_hbm, dst_hbm, out_hbm, src_v, dst_v,
                   u_sp, acc, isem, *bufs_sems):
        rows = bufs_sems[:NBUF]
        sems = bufs_sems[NBUF:]
        cid = lax.axis_index("c")
        sid = lax.axis_index("s")
        base = sid * RPT
        sl_mine = pl.ds(base, RPT)

        @pl.when(cid == 0)
        def _():
            pltpu.sync_copy(u0_hbm.at[sl_mine], u_sp.at[sl_mine])

        @pl.when(cid == 1)
        def _():
            pltpu.sync_copy(u1_hbm.at[sl_mine], u_sp.at[sl_mine])

        cbase = (cid * (NCH // 2) + sid * CHPT) if split_edges else sid * CHPT
        _zero_vmem(rows[0], K, W)
        for r in range(RPT // K):
            pltpu.sync_copy(rows[0], acc.at[pl.ds(base + r * K, K)])
        plsc.subcore_barrier()

        def fetch_idx(p, slot):
            pltpu.async_copy(
                src_hbm.at[pl.ds(cbase + p * NBUF, NBUF)], src_v.at[slot],
                isem)
            pltpu.async_copy(
                dst_hbm.at[pl.ds(cbase + p * NBUF, NBUF)], dst_v.at[slot],
                isem)

        def wait_idx(p, slot):
            pltpu.make_async_copy(
                src_hbm.at[pl.ds(cbase + p * NBUF, NBUF)], src_v.at[slot],
                isem).wait()
            pltpu.make_async_copy(
                dst_hbm.at[pl.ds(cbase + p * NBUF, NBUF)], dst_v.at[slot],
                isem).wait()

        NG = CHPT // NBUF
        fetch_idx(0, 0)
        wait_idx(0, 0)
        fetch_idx(1, 1)
        for b in range(NBUF):
            pltpu.async_copy(u_sp.at[src_v.at[0, b]], rows[b], sems[b])

        def group(p, _):
            slot = lax.rem(p, 2)
            nslot = lax.rem(p + 1, 2)
            for b in range(NBUF):
                pltpu.make_async_copy(u_sp.at[src_v.at[slot, b]], rows[b],
                                      sems[b]).wait()
                pltpu.sync_copy(rows[b], acc.at[dst_v.at[slot, b]], add=True)
                if b == 0:
                    @pl.when(p + 1 < NG)
                    def _():
                        wait_idx(p + 1, nslot)

                @pl.when(p + 1 < NG)
                def _():
                    pltpu.async_copy(u_sp.at[src_v.at[nslot, b]], rows[b],
                                     sems[b])

            @pl.when(p + 2 < NG)
            def _():
                fetch_idx(p + 2, slot)
            return 0
        lax.fori_loop(0, NG, group, 0)
        plsc.subcore_barrier()
        for r in range(RPT // K):
            sl = pl.ds(base + r * K, K)
            pltpu.sync_copy(acc.at[sl], out_hbm.at[cid, sl])

    return agg_kernel


def _matmul1(x, w1):
    BR = 1000

    def body(x_ref, w_ref, o_ref):
        o_ref[...] = lax.dot_general(
            x_ref[...], w_ref[...], (((1,), (1,)), ((), ())),
            preferred_element_type=jnp.float32)

    return pl.pallas_call(
        body,
        grid=(N // BR,),
        in_specs=[pl.BlockSpec((BR, 128), lambda i: (i, 0)),
                  pl.BlockSpec((128, 128), lambda i: (0, 0))],
        out_specs=pl.BlockSpec((BR, 128), lambda i: (i, 0)),
        out_shape=jax.ShapeDtypeStruct((N, 128), jnp.float32),
    )(x, w1)


def _dinv_u1(d0, d1, h):
    BR = 1000

    def body(d0_ref, d1_ref, h_ref, dinv_ref, u_ref):
        deg = d0_ref[...] + d1_ref[...] + 1.0
        dinv = lax.rsqrt(deg)
        dinv_ref[...] = dinv
        u_ref[...] = h_ref[...] * dinv[:, 0:1]

    return pl.pallas_call(
        body,
        grid=(N // BR,),
        in_specs=[pl.BlockSpec((BR, 8), lambda i: (i, 0)),
                  pl.BlockSpec((BR, 8), lambda i: (i, 0)),
                  pl.BlockSpec((BR, 128), lambda i: (i, 0))],
        out_specs=[pl.BlockSpec((BR, 8), lambda i: (i, 0)),
                   pl.BlockSpec((BR, 128), lambda i: (i, 0))],
        out_shape=[jax.ShapeDtypeStruct((N, 8), jnp.float32),
                   jax.ShapeDtypeStruct((N, 128), jnp.float32)],
    )(d0, d1, h)


def _z_stats(s0, s1, u1, dinv, b1):
    BR = 1000
    G = N // BR

    def body(s0_ref, s1_ref, u_ref, d_ref, b_ref, z_ref, sum_ref, ssq_ref):
        s = jnp.concatenate([s0_ref[...], s1_ref[...]], axis=1)
        z = d_ref[...][:, 0:1] * (s + u_ref[...]) + b_ref[...]
        z_ref[...] = z
        sum_ref[...] = jnp.sum(z, axis=0).reshape(1, 1, 128)
        ssq_ref[...] = jnp.sum(z * z, axis=0).reshape(1, 1, 128)

    return pl.pallas_call(
        body,
        grid=(G,),
        in_specs=[pl.BlockSpec((BR, 64), lambda i: (i, 0)),
                  pl.BlockSpec((BR, 64), lambda i: (i, 0)),
                  pl.BlockSpec((BR, 128), lambda i: (i, 0)),
                  pl.BlockSpec((BR, 8), lambda i: (i, 0)),
                  pl.BlockSpec((1, 128), lambda i: (0, 0))],
        out_specs=[pl.BlockSpec((BR, 128), lambda i: (i, 0)),
                   pl.BlockSpec((1, 1, 128), lambda i: (i, 0, 0)),
                   pl.BlockSpec((1, 1, 128), lambda i: (i, 0, 0))],
        out_shape=[jax.ShapeDtypeStruct((N, 128), jnp.float32),
                   jax.ShapeDtypeStruct((G, 1, 128), jnp.float32),
                   jax.ShapeDtypeStruct((G, 1, 128), jnp.float32)],
    )(s0, s1, u1, dinv, b1)


def _bn_relu_mm2(z, sums, ssq, gamma, beta, dinv, w2):
    BR = 1000
    G = N // BR

    def body(z_ref, sum_ref, ssq_ref, g_ref, b_ref, d_ref, w_ref, u2_ref):
        mean = jnp.sum(sum_ref[...], axis=0, keepdims=True) * (1.0 / N)
        var = jnp.sum(ssq_ref[...], axis=0, keepdims=True) * (1.0 / N) - mean * mean
        zb = (z_ref[...] - mean) * lax.rsqrt(var + 1e-5) * g_ref[...] + b_ref[...]
        h1 = jnp.maximum(zb, 0.0)
        u2_ref[...] = d_ref[...][:, 0:1] * lax.dot_general(
            h1, w_ref[...], (((1,), (1,)), ((), ())),
            preferred_element_type=jnp.float32)

    return pl.pallas_call(
        body,
        grid=(G,),
        in_specs=[pl.BlockSpec((BR, 128), lambda i: (i, 0)),
                  pl.BlockSpec((G, 128), lambda i: (0, 0)),
                  pl.BlockSpec((G, 128), lambda i: (0, 0)),
                  pl.BlockSpec((1, 128), lambda i: (0, 0)),
                  pl.BlockSpec((1, 128), lambda i: (0, 0)),
                  pl.BlockSpec((BR, 8), lambda i: (i, 0)),
                  pl.BlockSpec((64, 128), lambda i: (0, 0))],
        out_specs=pl.BlockSpec((BR, 64), lambda i: (i, 0)),
        out_shape=jax.ShapeDtypeStruct((N, 64), jnp.float32),
    )(z, sums, ssq, gamma, beta, dinv, w2)


def _final(s0, s1, u2, dinv, b2):
    BR = 1000

    def body(s0_ref, s1_ref, u_ref, d_ref, b_ref, o_ref):
        o_ref[...] = d_ref[...][:, 0:1] * (
            s0_ref[...] + s1_ref[...] + u_ref[...]) + b_ref[...]

    return pl.pallas_call(
        body,
        grid=(N // BR,),
        in_specs=[pl.BlockSpec((BR, 64), lambda i: (i, 0)),
                  pl.BlockSpec((BR, 64), lambda i: (i, 0)),
                  pl.BlockSpec((BR, 64), lambda i: (i, 0)),
                  pl.BlockSpec((BR, 8), lambda i: (i, 0)),
                  pl.BlockSpec((1, 64), lambda i: (0, 0))],
        out_specs=pl.BlockSpec((BR, 64), lambda i: (i, 0)),
        out_shape=jax.ShapeDtypeStruct((N, 64), jnp.float32),
    )(s0, s1, u2, dinv, b2)


def kernel(x, edge_index, W1, b1, gamma1, beta1, W2, b2):
    ei = edge_index.astype(jnp.int32)
    src, dst = ei[0], ei[1]
    pad = NCH * K - E
    srcp = jnp.concatenate(
        [src, jnp.zeros((pad,), jnp.int32)]).reshape(NCH, K)
    dstp = jnp.concatenate(
        [dst, jnp.full((pad,), N, jnp.int32)]).reshape(NCH, K)

    degp = _make_sc_deg()(dstp)
    h = _matmul1(x, W1)
    dinv, u1 = _dinv_u1(degp[0, :N, :8], degp[1, :N, :8], h)
    zpad = jnp.zeros((NPAD - N, 64), jnp.float32)
    u1p = jnp.concatenate([u1, jnp.zeros((NPAD - N, 128), jnp.float32)])
    s1p = _make_sc_agg(False)(u1p[:, :64], u1p[:, 64:], srcp, dstp)
    z, sums, ssq = _z_stats(s1p[0, :N], s1p[1, :N], u1, dinv,
                            b1.reshape(1, 128))
    sums = sums.reshape(-1, 128)
    ssq = ssq.reshape(-1, 128)
    u2 = _bn_relu_mm2(z, sums, ssq, gamma1.reshape(1, 128),
                      beta1.reshape(1, 128), dinv, W2)
    u2p = jnp.concatenate([u2, zpad])
    s2p = _make_sc_agg(True)(u2p, u2p, srcp, dstp)
    out = _final(s2p[0, :N], s2p[1, :N], u2, dinv, b2.reshape(1, 64))
    return out


# speedup vs baseline: 26.1834x; 1.0097x over previous
import functools

import jax
import jax.numpy as jnp
from jax import lax
from jax.experimental import pallas as pl
from jax.experimental.pallas import tpu as pltpu
from jax.experimental.pallas import tpu_sc as plsc

N = 10000
E = 320000
NPAD = 10240
NW = 32
K = 128
NCH = 2560
CH = NCH // NW
RPT = NPAD // 16


def _zero_vmem(buf, rows, d):
    def zrow(i, _):
        def zcol(j, _):
            buf[i, pl.ds(j * 16, 16)] = jnp.zeros((16,), jnp.float32)
            return 0
        return lax.fori_loop(0, d // 16, zcol, 0)
    lax.fori_loop(0, rows, zrow, 0)


def _make_sc_deg():
    mesh = plsc.VectorSubcoreMesh(core_axis_name="c", subcore_axis_name="s")

    @functools.partial(
        pl.kernel, mesh=mesh,
        compiler_params=pltpu.CompilerParams(use_tc_tiling_on_sc=False),
        out_type=jax.ShapeDtypeStruct((2, NPAD, 16), jnp.float32),
        scratch_types=[
            pltpu.VMEM((CH, K), jnp.int32),
            pltpu.VMEM((K, 16), jnp.float32),
            pltpu.VMEM((K, 16), jnp.float32),
            pltpu.VMEM_SHARED((NPAD, 16), jnp.float32),
        ],
    )
    def deg_kernel(dst_hbm, out_hbm, dst_v, ones_v, zero_v, acc):
        cid = lax.axis_index("c")
        sid = lax.axis_index("s")
        wid = cid * 16 + sid
        pltpu.sync_copy(dst_hbm.at[pl.ds(wid * CH, CH)], dst_v)

        def orow(i, _):
            ones_v[i, pl.ds(0, 16)] = jnp.ones((16,), jnp.float32)
            zero_v[i, pl.ds(0, 16)] = jnp.zeros((16,), jnp.float32)
            return 0
        lax.fori_loop(0, K, orow, 0)
        base = sid * RPT
        for r in range(RPT // K):
            pltpu.sync_copy(zero_v, acc.at[pl.ds(base + r * K, K)])
        plsc.subcore_barrier()

        def chunk(c, _):
            pltpu.sync_copy(ones_v, acc.at[dst_v.at[c]], add=True)
            return 0
        lax.fori_loop(0, CH, chunk, 0)
        plsc.subcore_barrier()
        for r in range(RPT // K):
            sl = pl.ds(base + r * K, K)
            pltpu.sync_copy(acc.at[sl], out_hbm.at[cid, sl])

    return deg_kernel


def _make_sc_agg(split_edges):
    mesh = plsc.VectorSubcoreMesh(core_axis_name="c", subcore_axis_name="s")

    W = 64
    NBUF = 4
    CHPT = (NCH // NW) if split_edges else (NCH // 16)

    @functools.partial(
        pl.kernel, mesh=mesh,
        compiler_params=pltpu.CompilerParams(use_tc_tiling_on_sc=False),
        out_type=jax.ShapeDtypeStruct((2, NPAD, W), jnp.float32),
        scratch_types=[
            pltpu.VMEM((2, NBUF, K), jnp.int32),
            pltpu.VMEM((2, NBUF, K), jnp.int32),
            pltpu.VMEM_SHARED((NPAD, W), jnp.float32),
            pltpu.VMEM_SHARED((NPAD, W), jnp.float32),
            pltpu.SemaphoreType.DMA,
        ] + [pltpu.VMEM((K, W), jnp.float32) for _ in range(NBUF)]
          + [pltpu.SemaphoreType.DMA for _ in range(NBUF)],
    )
    def agg_kernel(u0_hbm, u1_hbm, src_hbm, dst_hbm, out_hbm, src_v, dst_v,
                   u_sp, acc, isem, *bufs_sems):
        rows = bufs_sems[:NBUF]
        sems = bufs_sems[NBUF:]
        cid = lax.axis_index("c")
        sid = lax.axis_index("s")
        base = sid * RPT
        sl_mine = pl.ds(base, RPT)

        @pl.when(cid == 0)
        def _():
            pltpu.sync_copy(u0_hbm.at[sl_mine], u_sp.at[sl_mine])

        @pl.when(cid == 1)
        def _():
            pltpu.sync_copy(u1_hbm.at[sl_mine], u_sp.at[sl_mine])

        cbase = (cid * (NCH // 2) + sid * CHPT) if split_edges else sid * CHPT
        _zero_vmem(rows[0], K, W)
        for r in range(RPT // K):
            pltpu.sync_copy(rows[0], acc.at[pl.ds(base + r * K, K)])
        plsc.subcore_barrier()

        def fetch_idx(p, slot):
            pltpu.async_copy(
                src_hbm.at[pl.ds(cbase + p * NBUF, NBUF)], src_v.at[slot],
                isem)
            pltpu.async_copy(
                dst_hbm.at[pl.ds(cbase + p * NBUF, NBUF)], dst_v.at[slot],
                isem)

        def wait_idx(p, slot):
            pltpu.make_async_copy(
                src_hbm.at[pl.ds(cbase + p * NBUF, NBUF)], src_v.at[slot],
                isem).wait()
            pltpu.make_async_copy(
                dst_hbm.at[pl.ds(cbase + p * NBUF, NBUF)], dst_v.at[slot],
                isem).wait()

        NG = CHPT // NBUF
        fetch_idx(0, 0)
        wait_idx(0, 0)
        fetch_idx(1, 1)
        for b in range(NBUF):
            pltpu.async_copy(u_sp.at[src_v.at[0, b]], rows[b], sems[b])

        def group(p, _):
            slot = lax.rem(p, 2)
            nslot = lax.rem(p + 1, 2)
            for b in range(NBUF):
                pltpu.make_async_copy(u_sp.at[src_v.at[slot, b]], rows[b],
                                      sems[b]).wait()
                pltpu.sync_copy(rows[b], acc.at[dst_v.at[slot, b]], add=True)
                if b == 0:
                    @pl.when(p + 1 < NG)
                    def _():
                        wait_idx(p + 1, nslot)

                @pl.when(p + 1 < NG)
                def _():
                    pltpu.async_copy(u_sp.at[src_v.at[nslot, b]], rows[b],
                                     sems[b])

            @pl.when(p + 2 < NG)
            def _():
                fetch_idx(p + 2, slot)
            return 0
        lax.fori_loop(0, NG, group, 0)
        plsc.subcore_barrier()
        for r in range(RPT // K):
            sl = pl.ds(base + r * K, K)
            pltpu.sync_copy(acc.at[sl], out_hbm.at[cid, sl])

    return agg_kernel


def _matmul1(x, w1):
    BR = 1000

    def body(x_ref, w_ref, o_ref):
        o_ref[...] = lax.dot_general(
            x_ref[...], w_ref[...], (((1,), (1,)), ((), ())),
            preferred_element_type=jnp.float32)

    return pl.pallas_call(
        body,
        grid=(N // BR,),
        in_specs=[pl.BlockSpec((BR, 128), lambda i: (i, 0)),
                  pl.BlockSpec((128, 128), lambda i: (0, 0))],
        out_specs=pl.BlockSpec((BR, 128), lambda i: (i, 0)),
        out_shape=jax.ShapeDtypeStruct((N, 128), jnp.float32),
    )(x, w1)


def _dinv_u1(d0, d1, h):
    BR = 1000

    def body(d0_ref, d1_ref, h_ref, dinv_ref, u_ref):
        deg = d0_ref[...] + d1_ref[...] + 1.0
        dinv = lax.rsqrt(deg)
        dinv_ref[...] = dinv
        u_ref[...] = h_ref[...] * dinv[:, 0:1]

    return pl.pallas_call(
        body,
        grid=(N // BR,),
        in_specs=[pl.BlockSpec((BR, 8), lambda i: (i, 0)),
                  pl.BlockSpec((BR, 8), lambda i: (i, 0)),
                  pl.BlockSpec((BR, 128), lambda i: (i, 0))],
        out_specs=[pl.BlockSpec((BR, 8), lambda i: (i, 0)),
                   pl.BlockSpec((BR, 128), lambda i: (i, 0))],
        out_shape=[jax.ShapeDtypeStruct((N, 8), jnp.float32),
                   jax.ShapeDtypeStruct((N, 128), jnp.float32)],
    )(d0, d1, h)


def _z_stats(s0, s1, u1, dinv, b1):
    BR = 1000
    G = N // BR

    def body(s0_ref, s1_ref, u_ref, d_ref, b_ref, z_ref, sum_ref, ssq_ref):
        s = jnp.concatenate([s0_ref[...], s1_ref[...]], axis=1)
        z = d_ref[...][:, 0:1] * (s + u_ref[...]) + b_ref[...]
        z_ref[...] = z
        sum_ref[...] = jnp.sum(z, axis=0).reshape(1, 1, 128)
        ssq_ref[...] = jnp.sum(z * z, axis=0).reshape(1, 1, 128)

    return pl.pallas_call(
        body,
        grid=(G,),
        in_specs=[pl.BlockSpec((BR, 64), lambda i: (i, 0)),
                  pl.BlockSpec((BR, 64), lambda i: (i, 0)),
                  pl.BlockSpec((BR, 128), lambda i: (i, 0)),
                  pl.BlockSpec((BR, 8), lambda i: (i, 0)),
                  pl.BlockSpec((1, 128), lambda i: (0, 0))],
        out_specs=[pl.BlockSpec((BR, 128), lambda i: (i, 0)),
                   pl.BlockSpec((1, 1, 128), lambda i: (i, 0, 0)),
                   pl.BlockSpec((1, 1, 128), lambda i: (i, 0, 0))],
        out_shape=[jax.ShapeDtypeStruct((N, 128), jnp.float32),
                   jax.ShapeDtypeStruct((G, 1, 128), jnp.float32),
                   jax.ShapeDtypeStruct((G, 1, 128), jnp.float32)],
    )(s0, s1, u1, dinv, b1)


def _bn_relu_mm2(z, sums, ssq, gamma, beta, dinv, w2):
    BR = 1000
    G = N // BR

    def body(z_ref, sum_ref, ssq_ref, g_ref, b_ref, d_ref, w_ref, u2_ref):
        mean = jnp.sum(sum_ref[...], axis=0, keepdims=True) * (1.0 / N)
        var = jnp.sum(ssq_ref[...], axis=0, keepdims=True) * (1.0 / N) - mean * mean
        zb = (z_ref[...] - mean) * lax.rsqrt(var + 1e-5) * g_ref[...] + b_ref[...]
        h1 = jnp.maximum(zb, 0.0)
        u2_ref[...] = d_ref[...][:, 0:1] * lax.dot_general(
            h1, w_ref[...], (((1,), (1,)), ((), ())),
            preferred_element_type=jnp.float32)

    return pl.pallas_call(
        body,
        grid=(G,),
        in_specs=[pl.BlockSpec((BR, 128), lambda i: (i, 0)),
                  pl.BlockSpec((G, 128), lambda i: (0, 0)),
                  pl.BlockSpec((G, 128), lambda i: (0, 0)),
                  pl.BlockSpec((1, 128), lambda i: (0, 0)),
                  pl.BlockSpec((1, 128), lambda i: (0, 0)),
                  pl.BlockSpec((BR, 8), lambda i: (i, 0)),
                  pl.BlockSpec((64, 128), lambda i: (0, 0))],
        out_specs=pl.BlockSpec((BR, 64), lambda i: (i, 0)),
        out_shape=jax.ShapeDtypeStruct((N, 64), jnp.float32),
    )(z, sums, ssq, gamma, beta, dinv, w2)


def _final(s0, s1, u2, dinv, b2):
    BR = 1000

    def body(s0_ref, s1_ref, u_ref, d_ref, b_ref, o_ref):
        o_ref[...] = d_ref[...][:, 0:1] * (
            s0_ref[...] + s1_ref[...] + u_ref[...]) + b_ref[...]

    return pl.pallas_call(
        body,
        grid=(N // BR,),
        in_specs=[pl.BlockSpec((BR, 64), lambda i: (i, 0)),
                  pl.BlockSpec((BR, 64), lambda i: (i, 0)),
                  pl.BlockSpec((BR, 64), lambda i: (i, 0)),
                  pl.BlockSpec((BR, 8), lambda i: (i, 0)),
                  pl.BlockSpec((1, 64), lambda i: (0, 0))],
        out_specs=pl.BlockSpec((BR, 64), lambda i: (i, 0)),
        out_shape=jax.ShapeDtypeStruct((N, 64), jnp.float32),
    )(s0, s1, u2, dinv, b2)


def kernel(x, edge_index, W1, b1, gamma1, beta1, W2, b2):
    ei = edge_index.astype(jnp.int32)
    src, dst = ei[0], ei[1]
    pad = NCH * K - E
    srcp = jnp.concatenate(
        [src, jnp.zeros((pad,), jnp.int32)]).reshape(NCH, K)
    dstp = jnp.concatenate(
        [dst, jnp.full((pad,), N, jnp.int32)]).reshape(NCH, K)

    degp = _make_sc_deg()(dstp)
    h = _matmul1(x, W1)
    dinv, u1 = _dinv_u1(degp[0, :N, :8], degp[1, :N, :8], h)
    zpad = jnp.zeros((NPAD - N, 64), jnp.float32)
    u1p = jnp.concatenate([u1, jnp.zeros((NPAD - N, 128), jnp.float32)])
    s1p = _make_sc_agg(False)(u1p[:, :64], u1p[:, 64:], srcp, dstp)
    z, sums, ssq = _z_stats(s1p[0, :N], s1p[1, :N], u1, dinv,
                            b1.reshape(1, 128))
    sums = sums.reshape(-1, 128)
    ssq = ssq.reshape(-1, 128)
    u2 = _bn_relu_mm2(z, sums, ssq, gamma1.reshape(1, 128),
                      beta1.reshape(1, 128), dinv, W2)
    u2p = jnp.concatenate([u2, zpad])
    s2p = _make_sc_agg(True)(u2p, u2p, srcp, dstp)
    out = _final(s2p[0, :N], s2p[1, :N], u2, dinv, b2.reshape(1, 64))
    return out


# speedup vs baseline: 26.1906x; 1.0003x over previous
import functools

import jax
import jax.numpy as jnp
from jax import lax
from jax.experimental import pallas as pl
from jax.experimental.pallas import tpu as pltpu
from jax.experimental.pallas import tpu_sc as plsc

N = 10000
E = 320000
NPAD = 10240
NW = 32
K = 128
NCH = 2560
CH = NCH // NW
RPT = NPAD // 16


def _zero_vmem(buf, rows, d):
    def zrow(i, _):
        def zcol(j, _):
            buf[i, pl.ds(j * 16, 16)] = jnp.zeros((16,), jnp.float32)
            return 0
        return lax.fori_loop(0, d // 16, zcol, 0)
    lax.fori_loop(0, rows, zrow, 0)


def _make_sc_deg():
    mesh = plsc.VectorSubcoreMesh(core_axis_name="c", subcore_axis_name="s")

    @functools.partial(
        pl.kernel, mesh=mesh,
        compiler_params=pltpu.CompilerParams(use_tc_tiling_on_sc=False),
        out_type=jax.ShapeDtypeStruct((2, NPAD, 16), jnp.float32),
        scratch_types=[
            pltpu.VMEM((CH, K), jnp.int32),
            pltpu.VMEM((K, 16), jnp.float32),
            pltpu.VMEM((K, 16), jnp.float32),
            pltpu.VMEM_SHARED((NPAD, 16), jnp.float32),
        ],
    )
    def deg_kernel(dst_hbm, out_hbm, dst_v, ones_v, zero_v, acc):
        cid = lax.axis_index("c")
        sid = lax.axis_index("s")
        wid = cid * 16 + sid
        pltpu.sync_copy(dst_hbm.at[pl.ds(wid * CH, CH)], dst_v)

        def orow(i, _):
            ones_v[i, pl.ds(0, 16)] = jnp.ones((16,), jnp.float32)
            zero_v[i, pl.ds(0, 16)] = jnp.zeros((16,), jnp.float32)
            return 0
        lax.fori_loop(0, K, orow, 0)
        base = sid * RPT
        for r in range(RPT // K):
            pltpu.sync_copy(zero_v, acc.at[pl.ds(base + r * K, K)])
        plsc.subcore_barrier()

        def chunk(c, _):
            pltpu.sync_copy(ones_v, acc.at[dst_v.at[c]], add=True)
            return 0
        lax.fori_loop(0, CH, chunk, 0)
        plsc.subcore_barrier()
        for r in range(RPT // K):
            sl = pl.ds(base + r * K, K)
            pltpu.sync_copy(acc.at[sl], out_hbm.at[cid, sl])

    return deg_kernel


def _make_sc_agg(split_edges):
    mesh = plsc.VectorSubcoreMesh(core_axis_name="c", subcore_axis_name="s")

    W = 64
    NBUF = 4
    CHPT = (NCH // NW) if split_edges else (NCH // 16)

    @functools.partial(
        pl.kernel, mesh=mesh,
        compiler_params=pltpu.CompilerParams(use_tc_tiling_on_sc=False),
        out_type=jax.ShapeDtypeStruct((2, NPAD, W), jnp.float32),
        scratch_types=[
            pltpu.VMEM((2, NBUF, K), jnp.int32),
            pltpu.VMEM((2, NBUF, K), jnp.int32),
            pltpu.VMEM_SHARED((NPAD, W), jnp.float32),
            pltpu.VMEM_SHARED((NPAD, W), jnp.float32),
            pltpu.SemaphoreType.DMA,
        ] + [pltpu.VMEM((K, W), jnp.float32) for _ in range(NBUF)]
          + [pltpu.SemaphoreType.DMA for _ in range(NBUF)],
    )
    def agg_kernel(u0_hbm, u1_hbm, src_hbm, dst_hbm, out_hbm, src_v, dst_v,
                   u_sp, acc, isem, *bufs_sems):
        rows = bufs_sems[:NBUF]
        sems = bufs_sems[NBUF:]
        cid = lax.axis_index("c")
        sid = lax.axis_index("s")
        base = sid * RPT
        sl_mine = pl.ds(base, RPT)

        @pl.when(cid == 0)
        def _():
            pltpu.sync_copy(u0_hbm.at[sl_mine], u_sp.at[sl_mine])

        @pl.when(cid == 1)
        def _():
            pltpu.sync_copy(u1_hbm.at[sl_mine], u_sp.at[sl_mine])

        cbase = (cid * (NCH // 2) + sid * CHPT) if split_edges else sid * CHPT
        _zero_vmem(rows[0], K, W)
        for r in range(RPT // K):
            pltpu.sync_copy(rows[0], acc.at[pl.ds(base + r * K, K)])
        plsc.subcore_barrier()

        def fetch_idx(p, slot):
            pltpu.async_copy(
                src_hbm.at[pl.ds(cbase + p * NBUF, NBUF)], src_v.at[slot],
                isem)
            pltpu.async_copy(
                dst_hbm.at[pl.ds(cbase + p * NBUF, NBUF)], dst_v.at[slot],
                isem)

        def wait_idx(p, slot):
            pltpu.make_async_copy(
                src_hbm.at[pl.ds(cbase + p * NBUF, NBUF)], src_v.at[slot],
                isem).wait()
            pltpu.make_async_copy(
                dst_hbm.at[pl.ds(cbase + p * NBUF, NBUF)], dst_v.at[slot],
                isem).wait()

        NG = CHPT // NBUF
        fetch_idx(0, 0)
        wait_idx(0, 0)
        fetch_idx(1, 1)
        for b in range(NBUF):
            pltpu.async_copy(u_sp.at[src_v.at[0, b]], rows[b], sems[b])

        def group(p, _):
            slot = lax.rem(p, 2)
            nslot = lax.rem(p + 1, 2)
            for b in range(NBUF):
                pltpu.make_async_copy(u_sp.at[src_v.at[slot, b]], rows[b],
                                      sems[b]).wait()
                pltpu.sync_copy(rows[b], acc.at[dst_v.at[slot, b]], add=True)
                if b == 0:
                    @pl.when(p + 1 < NG)
                    def _():
                        wait_idx(p + 1, nslot)

                @pl.when(p + 1 < NG)
                def _():
                    pltpu.async_copy(u_sp.at[src_v.at[nslot, b]], rows[b],
                                     sems[b])

            @pl.when(p + 2 < NG)
            def _():
                fetch_idx(p + 2, slot)
            return 0
        lax.fori_loop(0, NG, group, 0)
        plsc.subcore_barrier()
        for r in range(RPT // K):
            sl = pl.ds(base + r * K, K)
            pltpu.sync_copy(acc.at[sl], out_hbm.at[cid, sl])

    return agg_kernel


def _mm1_dinv_u1(x, w1, d0, d1):
    BR = 1000

    def body(x_ref, w_ref, d0_ref, d1_ref, dinv_ref, u_ref):
        h = lax.dot_general(
            x_ref[...], w_ref[...], (((1,), (1,)), ((), ())),
            preferred_element_type=jnp.float32)
        deg = d0_ref[...] + d1_ref[...] + 1.0
        dinv = lax.rsqrt(deg)
        dinv_ref[...] = dinv
        u_ref[...] = h * dinv[:, 0:1]

    return pl.pallas_call(
        body,
        grid=(N // BR,),
        in_specs=[pl.BlockSpec((BR, 128), lambda i: (i, 0)),
                  pl.BlockSpec((128, 128), lambda i: (0, 0)),
                  pl.BlockSpec((BR, 8), lambda i: (i, 0)),
                  pl.BlockSpec((BR, 8), lambda i: (i, 0))],
        out_specs=[pl.BlockSpec((BR, 8), lambda i: (i, 0)),
                   pl.BlockSpec((BR, 128), lambda i: (i, 0))],
        out_shape=[jax.ShapeDtypeStruct((N, 8), jnp.float32),
                   jax.ShapeDtypeStruct((N, 128), jnp.float32)],
    )(x, w1, d0, d1)


def _z_stats(s0, s1, u1, dinv, b1):
    BR = 1000
    G = N // BR

    def body(s0_ref, s1_ref, u_ref, d_ref, b_ref, z_ref, sum_ref, ssq_ref):
        s = jnp.concatenate([s0_ref[...], s1_ref[...]], axis=1)
        z = d_ref[...][:, 0:1] * (s + u_ref[...]) + b_ref[...]
        z_ref[...] = z
        sum_ref[...] = jnp.sum(z, axis=0).reshape(1, 1, 128)
        ssq_ref[...] = jnp.sum(z * z, axis=0).reshape(1, 1, 128)

    return pl.pallas_call(
        body,
        grid=(G,),
        in_specs=[pl.BlockSpec((BR, 64), lambda i: (i, 0)),
                  pl.BlockSpec((BR, 64), lambda i: (i, 0)),
                  pl.BlockSpec((BR, 128), lambda i: (i, 0)),
                  pl.BlockSpec((BR, 8), lambda i: (i, 0)),
                  pl.BlockSpec((1, 128), lambda i: (0, 0))],
        out_specs=[pl.BlockSpec((BR, 128), lambda i: (i, 0)),
                   pl.BlockSpec((1, 1, 128), lambda i: (i, 0, 0)),
                   pl.BlockSpec((1, 1, 128), lambda i: (i, 0, 0))],
        out_shape=[jax.ShapeDtypeStruct((N, 128), jnp.float32),
                   jax.ShapeDtypeStruct((G, 1, 128), jnp.float32),
                   jax.ShapeDtypeStruct((G, 1, 128), jnp.float32)],
    )(s0, s1, u1, dinv, b1)


def _bn_relu_mm2(z, sums, ssq, gamma, beta, dinv, w2):
    BR = 1000
    G = N // BR

    def body(z_ref, sum_ref, ssq_ref, g_ref, b_ref, d_ref, w_ref, u2_ref):
        mean = jnp.sum(sum_ref[...], axis=0, keepdims=True) * (1.0 / N)
        var = jnp.sum(ssq_ref[...], axis=0, keepdims=True) * (1.0 / N) - mean * mean
        zb = (z_ref[...] - mean) * lax.rsqrt(var + 1e-5) * g_ref[...] + b_ref[...]
        h1 = jnp.maximum(zb, 0.0)
        u2_ref[...] = d_ref[...][:, 0:1] * lax.dot_general(
            h1, w_ref[...], (((1,), (1,)), ((), ())),
            preferred_element_type=jnp.float32)

    return pl.pallas_call(
        body,
        grid=(G,),
        in_specs=[pl.BlockSpec((BR, 128), lambda i: (i, 0)),
                  pl.BlockSpec((G, 128), lambda i: (0, 0)),
                  pl.BlockSpec((G, 128), lambda i: (0, 0)),
                  pl.BlockSpec((1, 128), lambda i: (0, 0)),
                  pl.BlockSpec((1, 128), lambda i: (0, 0)),
                  pl.BlockSpec((BR, 8), lambda i: (i, 0)),
                  pl.BlockSpec((64, 128), lambda i: (0, 0))],
        out_specs=pl.BlockSpec((BR, 64), lambda i: (i, 0)),
        out_shape=jax.ShapeDtypeStruct((N, 64), jnp.float32),
    )(z, sums, ssq, gamma, beta, dinv, w2)


def _final(s0, s1, u2, dinv, b2):
    BR = 1000

    def body(s0_ref, s1_ref, u_ref, d_ref, b_ref, o_ref):
        o_ref[...] = d_ref[...][:, 0:1] * (
            s0_ref[...] + s1_ref[...] + u_ref[...]) + b_ref[...]

    return pl.pallas_call(
        body,
        grid=(N // BR,),
        in_specs=[pl.BlockSpec((BR, 64), lambda i: (i, 0)),
                  pl.BlockSpec((BR, 64), lambda i: (i, 0)),
                  pl.BlockSpec((BR, 64), lambda i: (i, 0)),
                  pl.BlockSpec((BR, 8), lambda i: (i, 0)),
                  pl.BlockSpec((1, 64), lambda i: (0, 0))],
        out_specs=pl.BlockSpec((BR, 64), lambda i: (i, 0)),
        out_shape=jax.ShapeDtypeStruct((N, 64), jnp.float32),
    )(s0, s1, u2, dinv, b2)


def kernel(x, edge_index, W1, b1, gamma1, beta1, W2, b2):
    ei = edge_index.astype(jnp.int32)
    src, dst = ei[0], ei[1]
    pad = NCH * K - E
    srcp = jnp.concatenate(
        [src, jnp.zeros((pad,), jnp.int32)]).reshape(NCH, K)
    dstp = jnp.concatenate(
        [dst, jnp.full((pad,), N, jnp.int32)]).reshape(NCH, K)

    degp = _make_sc_deg()(dstp)
    dinv, u1 = _mm1_dinv_u1(x, W1, degp[0, :N, :8], degp[1, :N, :8])
    u1p = jnp.concatenate([u1, jnp.zeros((NPAD - N, 128), jnp.float32)])
    s1p = _make_sc_agg(False)(u1p[:, :64], u1p[:, 64:], srcp, dstp)
    z, sums, ssq = _z_stats(s1p[0], s1p[1], u1, dinv, b1.reshape(1, 128))
    sums = sums.reshape(-1, 128)
    ssq = ssq.reshape(-1, 128)
    u2 = _bn_relu_mm2(z, sums, ssq, gamma1.reshape(1, 128),
                      beta1.reshape(1, 128), dinv, W2)
    u2p = jnp.concatenate([u2, jnp.zeros((NPAD - N, 64), jnp.float32)])
    s2p = _make_sc_agg(True)(u2p, u2p, srcp, dstp)
    out = _final(s2p[0], s2p[1], u2, dinv, b2.reshape(1, 64))
    return out


# speedup vs baseline: 28.9684x; 1.1061x over previous
import functools

import jax
import jax.numpy as jnp
from jax import lax
from jax.experimental import pallas as pl
from jax.experimental.pallas import tpu as pltpu
from jax.experimental.pallas import tpu_sc as plsc

N = 10000
E = 320000
NPAD = 10240
NW = 32
K = 128
NCH = 2560
CH = NCH // NW
RPT = NPAD // 16


def _zero_vmem(buf, rows, d):
    def zrow(i, _):
        def zcol(j, _):
            buf[i, pl.ds(j * 16, 16)] = jnp.zeros((16,), jnp.float32)
            return 0
        return lax.fori_loop(0, d // 16, zcol, 0)
    lax.fori_loop(0, rows, zrow, 0)


def _make_sc_deg():
    mesh = plsc.VectorSubcoreMesh(core_axis_name="c", subcore_axis_name="s")

    @functools.partial(
        pl.kernel, mesh=mesh,
        compiler_params=pltpu.CompilerParams(use_tc_tiling_on_sc=False),
        out_type=jax.ShapeDtypeStruct((2, NPAD, 16), jnp.float32),
        scratch_types=[
            pltpu.VMEM((CH, K), jnp.int32),
            pltpu.VMEM((K, 16), jnp.float32),
            pltpu.VMEM((K, 16), jnp.float32),
            pltpu.VMEM_SHARED((NPAD, 16), jnp.float32),
        ],
    )
    def deg_kernel(dst_hbm, out_hbm, dst_v, ones_v, zero_v, acc):
        cid = lax.axis_index("c")
        sid = lax.axis_index("s")
        wid = cid * 16 + sid
        pltpu.sync_copy(dst_hbm.at[pl.ds(wid * CH, CH)], dst_v)

        def orow(i, _):
            ones_v[i, pl.ds(0, 16)] = jnp.ones((16,), jnp.float32)
            zero_v[i, pl.ds(0, 16)] = jnp.zeros((16,), jnp.float32)
            return 0
        lax.fori_loop(0, K, orow, 0)
        base = sid * RPT
        for r in range(RPT // K):
            pltpu.sync_copy(zero_v, acc.at[pl.ds(base + r * K, K)])
        plsc.subcore_barrier()

        def chunk(c, _):
            pltpu.sync_copy(ones_v, acc.at[dst_v.at[c]], add=True)
            return 0
        lax.fori_loop(0, CH, chunk, 0)
        plsc.subcore_barrier()
        for r in range(RPT // K):
            sl = pl.ds(base + r * K, K)
            pltpu.sync_copy(acc.at[sl], out_hbm.at[cid, sl])

    return deg_kernel


def _make_sc_agg(split_edges):
    mesh = plsc.VectorSubcoreMesh(core_axis_name="c", subcore_axis_name="s")

    W = 64
    NBUF = 4
    RING = 5
    GLEAD = 3
    CHPT = (NCH // NW) if split_edges else (NCH // 16)

    @functools.partial(
        pl.kernel, mesh=mesh,
        compiler_params=pltpu.CompilerParams(use_tc_tiling_on_sc=False),
        out_type=jax.ShapeDtypeStruct((2, NPAD, W), jnp.float32),
        scratch_types=[
            pltpu.VMEM((4, NBUF, K), jnp.int32),
            pltpu.VMEM((4, NBUF, K), jnp.int32),
            pltpu.VMEM_SHARED((NPAD, W), jnp.float32),
            pltpu.VMEM_SHARED((NPAD, W), jnp.float32),
            pltpu.SemaphoreType.DMA,
        ] + [pltpu.VMEM((K, W), jnp.float32) for _ in range(RING)]
          + [pltpu.SemaphoreType.DMA for _ in range(RING)]
          + [pltpu.SemaphoreType.DMA for _ in range(RING)],
    )
    def agg_kernel(u0_hbm, u1_hbm, src_hbm, dst_hbm, out_hbm, src_v, dst_v,
                   u_sp, acc, isem, *bufs_sems):
        rows = bufs_sems[:RING]
        sems = bufs_sems[RING:2 * RING]
        ssems = bufs_sems[2 * RING:]
        cid = lax.axis_index("c")
        sid = lax.axis_index("s")
        base = sid * RPT
        sl_mine = pl.ds(base, RPT)

        @pl.when(cid == 0)
        def _():
            pltpu.sync_copy(u0_hbm.at[sl_mine], u_sp.at[sl_mine])

        @pl.when(cid == 1)
        def _():
            pltpu.sync_copy(u1_hbm.at[sl_mine], u_sp.at[sl_mine])

        cbase = (cid * (NCH // 2) + sid * CHPT) if split_edges else sid * CHPT
        _zero_vmem(rows[0], K, W)
        for r in range(RPT // K):
            pltpu.sync_copy(rows[0], acc.at[pl.ds(base + r * K, K)])
        plsc.subcore_barrier()

        def fetch_idx(p, slot):
            pltpu.async_copy(
                src_hbm.at[pl.ds(cbase + p * NBUF, NBUF)], src_v.at[slot],
                isem)
            pltpu.async_copy(
                dst_hbm.at[pl.ds(cbase + p * NBUF, NBUF)], dst_v.at[slot],
                isem)

        def wait_idx(p, slot):
            pltpu.make_async_copy(
                src_hbm.at[pl.ds(cbase + p * NBUF, NBUF)], src_v.at[slot],
                isem).wait()
            pltpu.make_async_copy(
                dst_hbm.at[pl.ds(cbase + p * NBUF, NBUF)], dst_v.at[slot],
                isem).wait()

        NGRP = CHPT // NBUF
        fetch_idx(0, 0)
        wait_idx(0, 0)
        fetch_idx(1, 1)
        for b in range(GLEAD):
            pltpu.async_copy(u_sp.at[src_v.at[0, b]], rows[b], sems[b])

        def sidx(c):
            return lax.rem(c // NBUF, 4), lax.rem(c, NBUF)

        def drain_scatter(b):
            pltpu.make_async_copy(rows[b], acc.at[dst_v.at[0, 0]],
                                  ssems[b]).wait()

        def super_group(q, _):
            for b in range(RING):
                c = q * RING + b
                j = lax.rem(c, NBUF)
                p = c // NBUF

                @pl.when(jnp.logical_and(j == 0, p + 1 < NGRP))
                def _():
                    wait_idx(p + 1, lax.rem(p + 1, 4))

                @pl.when(jnp.logical_and(j == 0, p + 2 < NGRP))
                def _():
                    fetch_idx(p + 2, lax.rem(p + 2, 4))

                slot, jj = sidx(c)
                pltpu.make_async_copy(u_sp.at[src_v.at[slot, jj]], rows[b],
                                      sems[b]).wait()
                pltpu.async_copy(rows[b], acc.at[dst_v.at[slot, jj]],
                                 ssems[b], add=True)
                n = c + GLEAD
                bn = (b + GLEAD) % RING

                @pl.when(n < CHPT)
                def _():
                    @pl.when(c >= RING - GLEAD)
                    def _():
                        drain_scatter(bn)
                    nslot, nj = sidx(n)
                    pltpu.async_copy(u_sp.at[src_v.at[nslot, nj]], rows[bn],
                                     sems[bn])
            return 0
        lax.fori_loop(0, CHPT // RING, super_group, 0)
        for b in range(RING):
            drain_scatter(b)
        plsc.subcore_barrier()
        for r in range(RPT // K):
            sl = pl.ds(base + r * K, K)
            pltpu.sync_copy(acc.at[sl], out_hbm.at[cid, sl])

    return agg_kernel


def _mm1_dinv_u1(x, w1, d0, d1):
    BR = 1000

    def body(x_ref, w_ref, d0_ref, d1_ref, dinv_ref, u_ref):
        h = lax.dot_general(
            x_ref[...], w_ref[...], (((1,), (1,)), ((), ())),
            preferred_element_type=jnp.float32)
        deg = d0_ref[...] + d1_ref[...] + 1.0
        dinv = lax.rsqrt(deg)
        dinv_ref[...] = dinv
        u_ref[...] = h * dinv[:, 0:1]

    return pl.pallas_call(
        body,
        grid=(N // BR,),
        in_specs=[pl.BlockSpec((BR, 128), lambda i: (i, 0)),
                  pl.BlockSpec((128, 128), lambda i: (0, 0)),
                  pl.BlockSpec((BR, 8), lambda i: (i, 0)),
                  pl.BlockSpec((BR, 8), lambda i: (i, 0))],
        out_specs=[pl.BlockSpec((BR, 8), lambda i: (i, 0)),
                   pl.BlockSpec((BR, 128), lambda i: (i, 0))],
        out_shape=[jax.ShapeDtypeStruct((N, 8), jnp.float32),
                   jax.ShapeDtypeStruct((N, 128), jnp.float32)],
    )(x, w1, d0, d1)


def _z_stats(s0, s1, u1, dinv, b1):
    BR = 1000
    G = N // BR

    def body(s0_ref, s1_ref, u_ref, d_ref, b_ref, z_ref, sum_ref, ssq_ref):
        s = jnp.concatenate([s0_ref[...], s1_ref[...]], axis=1)
        z = d_ref[...][:, 0:1] * (s + u_ref[...]) + b_ref[...]
        z_ref[...] = z
        sum_ref[...] = jnp.sum(z, axis=0).reshape(1, 1, 128)
        ssq_ref[...] = jnp.sum(z * z, axis=0).reshape(1, 1, 128)

    return pl.pallas_call(
        body,
        grid=(G,),
        in_specs=[pl.BlockSpec((BR, 64), lambda i: (i, 0)),
                  pl.BlockSpec((BR, 64), lambda i: (i, 0)),
                  pl.BlockSpec((BR, 128), lambda i: (i, 0)),
                  pl.BlockSpec((BR, 8), lambda i: (i, 0)),
                  pl.BlockSpec((1, 128), lambda i: (0, 0))],
        out_specs=[pl.BlockSpec((BR, 128), lambda i: (i, 0)),
                   pl.BlockSpec((1, 1, 128), lambda i: (i, 0, 0)),
                   pl.BlockSpec((1, 1, 128), lambda i: (i, 0, 0))],
        out_shape=[jax.ShapeDtypeStruct((N, 128), jnp.float32),
                   jax.ShapeDtypeStruct((G, 1, 128), jnp.float32),
                   jax.ShapeDtypeStruct((G, 1, 128), jnp.float32)],
    )(s0, s1, u1, dinv, b1)


def _bn_relu_mm2(z, sums, ssq, gamma, beta, dinv, w2):
    BR = 1000
    G = N // BR

    def body(z_ref, sum_ref, ssq_ref, g_ref, b_ref, d_ref, w_ref, u2_ref):
        mean = jnp.sum(sum_ref[...], axis=0, keepdims=True) * (1.0 / N)
        var = jnp.sum(ssq_ref[...], axis=0, keepdims=True) * (1.0 / N) - mean * mean
        zb = (z_ref[...] - mean) * lax.rsqrt(var + 1e-5) * g_ref[...] + b_ref[...]
        h1 = jnp.maximum(zb, 0.0)
        u2_ref[...] = d_ref[...][:, 0:1] * lax.dot_general(
            h1, w_ref[...], (((1,), (1,)), ((), ())),
            preferred_element_type=jnp.float32)

    return pl.pallas_call(
        body,
        grid=(G,),
        in_specs=[pl.BlockSpec((BR, 128), lambda i: (i, 0)),
                  pl.BlockSpec((G, 128), lambda i: (0, 0)),
                  pl.BlockSpec((G, 128), lambda i: (0, 0)),
                  pl.BlockSpec((1, 128), lambda i: (0, 0)),
                  pl.BlockSpec((1, 128), lambda i: (0, 0)),
                  pl.BlockSpec((BR, 8), lambda i: (i, 0)),
                  pl.BlockSpec((64, 128), lambda i: (0, 0))],
        out_specs=pl.BlockSpec((BR, 64), lambda i: (i, 0)),
        out_shape=jax.ShapeDtypeStruct((N, 64), jnp.float32),
    )(z, sums, ssq, gamma, beta, dinv, w2)


def _final(s0, s1, u2, dinv, b2):
    BR = 1000

    def body(s0_ref, s1_ref, u_ref, d_ref, b_ref, o_ref):
        o_ref[...] = d_ref[...][:, 0:1] * (
            s0_ref[...] + s1_ref[...] + u_ref[...]) + b_ref[...]

    return pl.pallas_call(
        body,
        grid=(N // BR,),
        in_specs=[pl.BlockSpec((BR, 64), lambda i: (i, 0)),
                  pl.BlockSpec((BR, 64), lambda i: (i, 0)),
                  pl.BlockSpec((BR, 64), lambda i: (i, 0)),
                  pl.BlockSpec((BR, 8), lambda i: (i, 0)),
                  pl.BlockSpec((1, 64), lambda i: (0, 0))],
        out_specs=pl.BlockSpec((BR, 64), lambda i: (i, 0)),
        out_shape=jax.ShapeDtypeStruct((N, 64), jnp.float32),
    )(s0, s1, u2, dinv, b2)


def kernel(x, edge_index, W1, b1, gamma1, beta1, W2, b2):
    ei = edge_index.astype(jnp.int32)
    src, dst = ei[0], ei[1]
    pad = NCH * K - E
    srcp = jnp.concatenate(
        [src, jnp.zeros((pad,), jnp.int32)]).reshape(NCH, K)
    dstp = jnp.concatenate(
        [dst, jnp.full((pad,), N, jnp.int32)]).reshape(NCH, K)

    degp = _make_sc_deg()(dstp)
    dinv, u1 = _mm1_dinv_u1(x, W1, degp[0, :N, :8], degp[1, :N, :8])
    u1p = jnp.concatenate([u1, jnp.zeros((NPAD - N, 128), jnp.float32)])
    s1p = _make_sc_agg(False)(u1p[:, :64], u1p[:, 64:], srcp, dstp)
    z, sums, ssq = _z_stats(s1p[0], s1p[1], u1, dinv, b1.reshape(1, 128))
    sums = sums.reshape(-1, 128)
    ssq = ssq.reshape(-1, 128)
    u2 = _bn_relu_mm2(z, sums, ssq, gamma1.reshape(1, 128),
                      beta1.reshape(1, 128), dinv, W2)
    u2p = jnp.concatenate([u2, jnp.zeros((NPAD - N, 64), jnp.float32)])
    s2p = _make_sc_agg(True)(u2p, u2p, srcp, dstp)
    out = _final(s2p[0], s2p[1], u2, dinv, b2.reshape(1, 64))
    return out
